# Initial kernel scaffold; baseline (speedup 1.0000x reference)
import jax
import jax.numpy as jnp
from jax.experimental import pallas as pl


def kernel(x, pos, edge_index, W1, b1, W2, b2, ne_ln_g, ne_ln_b, convW, convB, conv_ln_g, conv_ln_b, ln_g, ln_b, Wg1, bg1, Wg2, bg2, Wc1, bc1, Wc2, bc2, Wp1, bp1, Wp2, bp2):
    raise NotImplementedError("write your pallas kernel here")


# speedup vs baseline: 1.0861x; 1.0861x over previous
import functools

import jax
import jax.numpy as jnp
from jax import lax
from jax.experimental import pallas as pl
from jax.experimental.pallas import tpu as pltpu

H = 128
GEOF = 32


def _silu(x):
    return x * jax.nn.sigmoid(x)


def _ln(x, g, b, eps=1e-5):
    m = jnp.mean(x, axis=-1, keepdims=True)
    v = jnp.mean((x - m) ** 2, axis=-1, keepdims=True)
    return (x - m) * jax.lax.rsqrt(v + eps) * g + b


def _enc_body(x_ref, w1_ref, b1_ref, w2_ref, b2_ref, g_ref, bb_ref, o_ref):
    h = jnp.dot(x_ref[...], w1_ref[...], preferred_element_type=jnp.float32)
    h = _silu(h + b1_ref[...])
    h = jnp.dot(h, w2_ref[...], preferred_element_type=jnp.float32) + b2_ref[...]
    o_ref[...] = _ln(h, g_ref[...], bb_ref[...])


def _encoder(x, W1, b1, W2, b2, g, b):
    N, D = x.shape
    BN = 1000
    grid = (N // BN,)
    return pl.pallas_call(
        _enc_body,
        grid=grid,
        in_specs=[
            pl.BlockSpec((BN, D), lambda i: (i, 0)),
            pl.BlockSpec((D, H), lambda i: (0, 0)),
            pl.BlockSpec((1, H), lambda i: (0, 0)),
            pl.BlockSpec((H, H), lambda i: (0, 0)),
            pl.BlockSpec((1, H), lambda i: (0, 0)),
            pl.BlockSpec((1, H), lambda i: (0, 0)),
            pl.BlockSpec((1, H), lambda i: (0, 0)),
        ],
        out_specs=pl.BlockSpec((BN, H), lambda i: (i, 0)),
        out_shape=jax.ShapeDtypeStruct((N, H), jnp.float32),
    )(x, W1, b1.reshape(1, H), W2, b2.reshape(1, H), g.reshape(1, H), b.reshape(1, H))


def _layer_body(seg_ref, geo_ref, h_ref, wh_ref, wg_ref, cg_ref, cb_ref,
                lg_ref, lb_ref, o_ref):
    seg = seg_ref[0] + seg_ref[1]
    agg = jnp.dot(seg, wh_ref[...], preferred_element_type=jnp.float32)
    agg = agg + jnp.dot(geo_ref[...], wg_ref[...], preferred_element_type=jnp.float32)
    h_new = _ln(_silu(agg), cg_ref[...], cb_ref[...])
    o_ref[...] = _ln(h_ref[...] + h_new, lg_ref[...], lb_ref[...])


def _layer_dense(seg2, geo, h, Wh, Wgeo, cg, cb, lg, lb):
    N = h.shape[0]
    BN = 1000
    grid = (N // BN,)
    return pl.pallas_call(
        _layer_body,
        grid=grid,
        in_specs=[
            pl.BlockSpec((2, BN, H), lambda i: (0, i, 0)),
            pl.BlockSpec((BN, GEOF), lambda i: (i, 0)),
            pl.BlockSpec((BN, H), lambda i: (i, 0)),
            pl.BlockSpec((H, H), lambda i: (0, 0)),
            pl.BlockSpec((GEOF, H), lambda i: (0, 0)),
            pl.BlockSpec((1, H), lambda i: (0, 0)),
            pl.BlockSpec((1, H), lambda i: (0, 0)),
            pl.BlockSpec((1, H), lambda i: (0, 0)),
            pl.BlockSpec((1, H), lambda i: (0, 0)),
        ],
        out_specs=pl.BlockSpec((BN, H), lambda i: (i, 0)),
        out_shape=jax.ShapeDtypeStruct((N, H), jnp.float32),
    )(seg2, geo, h, Wh, Wgeo, cg.reshape(1, H), cb.reshape(1, H),
      lg.reshape(1, H), lb.reshape(1, H))


def _readout_body(h_ref, wg1_ref, bg1_ref, wg2_ref, bg2_ref,
                  wc1_ref, bc1_ref, wc2_ref, bc2_ref,
                  wp1_ref, bp1_ref, wp2_ref, bp2_ref,
                  logits_ref, proj_ref, emb_ref,
                  m_s, s_s, v_s):
    i = pl.program_id(0)
    nb = pl.num_programs(0)
    h = h_ref[...]
    z = jnp.dot(_silu(jnp.dot(h, wg1_ref[...],
                              preferred_element_type=jnp.float32) + bg1_ref[...]),
                wg2_ref[...], preferred_element_type=jnp.float32) + bg2_ref[0, 0]
    bm = jnp.max(z)

    @pl.when(i == 0)
    def _():
        m_s[...] = jnp.full_like(m_s, -jnp.inf)
        s_s[...] = jnp.zeros_like(s_s)
        v_s[...] = jnp.zeros_like(v_s)

    m_old = m_s[0, 0]
    m_new = jnp.maximum(m_old, bm)
    scale = jnp.exp(m_old - m_new)
    w = jnp.exp(z - m_new)
    s_s[...] = s_s[...] * scale + jnp.sum(w)
    v_s[...] = v_s[...] * scale + jnp.sum(h * w, axis=0, keepdims=True)
    m_s[...] = jnp.full_like(m_s, m_new)

    @pl.when(i == nb - 1)
    def _():
        emb = v_s[...] / s_s[0, 0]
        emb_ref[...] = emb
        c = jnp.dot(_silu(jnp.dot(emb, wc1_ref[...],
                                  preferred_element_type=jnp.float32) + bc1_ref[...]),
                    wc2_ref[...], preferred_element_type=jnp.float32) + bc2_ref[...]
        logits_ref[...] = c
        p = jnp.dot(_silu(jnp.dot(emb, wp1_ref[...],
                                  preferred_element_type=jnp.float32) + bp1_ref[...]),
                    wp2_ref[...], preferred_element_type=jnp.float32) + bp2_ref[...]
        nrm = jnp.maximum(jnp.sqrt(jnp.sum(p * p)), 1e-12)
        proj_ref[...] = p / nrm


def _readout(h, Wg1, bg1, Wg2, bg2, Wc1, bc1, Wc2, bc2, Wp1, bp1, Wp2, bp2):
    N = h.shape[0]
    BN = 1000
    grid = (N // BN,)
    Hq = Wg1.shape[1]
    Hc = Wc1.shape[1]
    C = Wc2.shape[1]
    P = Wp2.shape[1]
    full = lambda r, c: pl.BlockSpec((r, c), lambda i: (0, 0))
    return pl.pallas_call(
        _readout_body,
        grid=grid,
        in_specs=[
            pl.BlockSpec((BN, H), lambda i: (i, 0)),
            full(H, Hq), full(1, Hq), full(Hq, 1), full(1, 1),
            full(H, Hc), full(1, Hc), full(Hc, C), full(1, C),
            full(H, H), full(1, H), full(H, P), full(1, P),
        ],
        out_specs=[full(1, C), full(1, P), full(1, H)],
        out_shape=[
            jax.ShapeDtypeStruct((1, C), jnp.float32),
            jax.ShapeDtypeStruct((1, P), jnp.float32),
            jax.ShapeDtypeStruct((1, H), jnp.float32),
        ],
        scratch_shapes=[
            pltpu.VMEM((1, 128), jnp.float32),
            pltpu.VMEM((1, 128), jnp.float32),
            pltpu.VMEM((1, H), jnp.float32),
        ],
    )(h, Wg1, bg1.reshape(1, Hq), Wg2, bg2.reshape(1, 1),
      Wc1, bc1.reshape(1, Hc), Wc2, bc2.reshape(1, C),
      Wp1, bp1.reshape(1, H), Wp2, bp2.reshape(1, P))


def _geo_aggregate(pos, src, dst, N):
    rel = pos[dst] - pos[src]
    r2 = jnp.sum(rel * rel, axis=-1, keepdims=True)
    d = jnp.sqrt(r2)
    cutoff, R = 10.0, 16
    centers = jnp.linspace(0.0, cutoff, R)
    width = 0.5 * (cutoff / R) ** 2
    rad = jnp.exp(-width * (d - centers[None, :]) ** 2)
    sh = rel / jnp.maximum(d, 1e-12)
    ones = jnp.ones((src.shape[0], 1), jnp.float32)
    zeros = jnp.zeros((src.shape[0], GEOF - 20), jnp.float32)
    feats = jnp.concatenate([sh, rad, ones, zeros], axis=-1)
    return jnp.zeros((N, GEOF), jnp.float32).at[dst].add(feats)


def _segsum(h, src, dst, N):
    E = src.shape[0]
    half = E // 2
    p0 = jnp.zeros((N, H), jnp.float32).at[dst[:half]].add(h[src[:half]])
    p1 = jnp.zeros((N, H), jnp.float32).at[dst[half:]].add(h[src[half:]])
    return jnp.stack([p0, p1])


def kernel(x, pos, edge_index, W1, b1, W2, b2, ne_ln_g, ne_ln_b, convW, convB,
           conv_ln_g, conv_ln_b, ln_g, ln_b, Wg1, bg1, Wg2, bg2, Wc1, bc1,
           Wc2, bc2, Wp1, bp1, Wp2, bp2):
    N = x.shape[0]
    L = convW.shape[0]
    src = edge_index[0]
    dst = edge_index[1]

    Wgeo = jnp.concatenate(
        [convW[:, H:H + 19, :], convB[:, None, :],
         jnp.zeros((L, GEOF - 20, H), jnp.float32)], axis=1)
    Wh = convW[:, :H, :]

    h = _encoder(x, W1, b1, W2, b2, ne_ln_g, ne_ln_b)
    geo = _geo_aggregate(pos, src, dst, N)

    for i in range(L):
        seg2 = _segsum(h, src, dst, N)
        h = _layer_dense(seg2, geo, h, Wh[i], Wgeo[i], conv_ln_g[i],
                         conv_ln_b[i], ln_g[i], ln_b[i])

    return _readout(h, Wg1, bg1, Wg2, bg2, Wc1, bc1, Wc2, bc2,
                    Wp1, bp1, Wp2, bp2)


# speedup vs baseline: 2.2754x; 2.0949x over previous
import functools

import jax
import jax.numpy as jnp
from jax import lax
from jax.experimental import pallas as pl
from jax.experimental.pallas import tpu as pltpu
from jax.experimental.pallas import tpu_sc as plsc

H = 128
GEOF = 32


def _silu(x):
    return x * jax.nn.sigmoid(x)


def _ln(x, g, b, eps=1e-5):
    m = jnp.mean(x, axis=-1, keepdims=True)
    v = jnp.mean((x - m) ** 2, axis=-1, keepdims=True)
    return (x - m) * jax.lax.rsqrt(v + eps) * g + b


def _enc_body(x_ref, w1_ref, b1_ref, w2_ref, b2_ref, g_ref, bb_ref, o_ref):
    h = jnp.dot(x_ref[...], w1_ref[...], preferred_element_type=jnp.float32)
    h = _silu(h + b1_ref[...])
    h = jnp.dot(h, w2_ref[...], preferred_element_type=jnp.float32) + b2_ref[...]
    o_ref[...] = _ln(h, g_ref[...], bb_ref[...])


def _encoder(x, W1, b1, W2, b2, g, b):
    N, D = x.shape
    BN = 1000
    grid = (N // BN,)
    return pl.pallas_call(
        _enc_body,
        grid=grid,
        in_specs=[
            pl.BlockSpec((BN, D), lambda i: (i, 0)),
            pl.BlockSpec((D, H), lambda i: (0, 0)),
            pl.BlockSpec((1, H), lambda i: (0, 0)),
            pl.BlockSpec((H, H), lambda i: (0, 0)),
            pl.BlockSpec((1, H), lambda i: (0, 0)),
            pl.BlockSpec((1, H), lambda i: (0, 0)),
            pl.BlockSpec((1, H), lambda i: (0, 0)),
        ],
        out_specs=pl.BlockSpec((BN, H), lambda i: (i, 0)),
        out_shape=jax.ShapeDtypeStruct((N, H), jnp.float32),
    )(x, W1, b1.reshape(1, H), W2, b2.reshape(1, H), g.reshape(1, H), b.reshape(1, H))


def _layer_body(seg_ref, geo_ref, h_ref, wh_ref, wg_ref, cg_ref, cb_ref,
                lg_ref, lb_ref, o_ref):
    seg = seg_ref[0] + seg_ref[1]
    agg = jnp.dot(seg, wh_ref[...], preferred_element_type=jnp.float32)
    agg = agg + jnp.dot(geo_ref[...], wg_ref[...], preferred_element_type=jnp.float32)
    h_new = _ln(_silu(agg), cg_ref[...], cb_ref[...])
    o_ref[...] = _ln(h_ref[...] + h_new, lg_ref[...], lb_ref[...])


def _layer_dense(seg2, geo, h, Wh, Wgeo, cg, cb, lg, lb):
    N = h.shape[0]
    BN = 1000
    grid = (N // BN,)
    return pl.pallas_call(
        _layer_body,
        grid=grid,
        in_specs=[
            pl.BlockSpec((2, BN, H), lambda i: (0, i, 0)),
            pl.BlockSpec((BN, GEOF), lambda i: (i, 0)),
            pl.BlockSpec((BN, H), lambda i: (i, 0)),
            pl.BlockSpec((H, H), lambda i: (0, 0)),
            pl.BlockSpec((GEOF, H), lambda i: (0, 0)),
            pl.BlockSpec((1, H), lambda i: (0, 0)),
            pl.BlockSpec((1, H), lambda i: (0, 0)),
            pl.BlockSpec((1, H), lambda i: (0, 0)),
            pl.BlockSpec((1, H), lambda i: (0, 0)),
        ],
        out_specs=pl.BlockSpec((BN, H), lambda i: (i, 0)),
        out_shape=jax.ShapeDtypeStruct((N, H), jnp.float32),
    )(seg2, geo, h, Wh, Wgeo, cg.reshape(1, H), cb.reshape(1, H),
      lg.reshape(1, H), lb.reshape(1, H))


def _readout_body(h_ref, wg1_ref, bg1_ref, wg2_ref, bg2_ref,
                  wc1_ref, bc1_ref, wc2_ref, bc2_ref,
                  wp1_ref, bp1_ref, wp2_ref, bp2_ref,
                  logits_ref, proj_ref, emb_ref,
                  m_s, s_s, v_s):
    i = pl.program_id(0)
    nb = pl.num_programs(0)
    h = h_ref[...]
    z = jnp.dot(_silu(jnp.dot(h, wg1_ref[...],
                              preferred_element_type=jnp.float32) + bg1_ref[...]),
                wg2_ref[...], preferred_element_type=jnp.float32) + bg2_ref[0, 0]
    bm = jnp.max(z)

    @pl.when(i == 0)
    def _():
        m_s[...] = jnp.full_like(m_s, -jnp.inf)
        s_s[...] = jnp.zeros_like(s_s)
        v_s[...] = jnp.zeros_like(v_s)

    m_old = m_s[0, 0]
    m_new = jnp.maximum(m_old, bm)
    scale = jnp.exp(m_old - m_new)
    w = jnp.exp(z - m_new)
    s_s[...] = s_s[...] * scale + jnp.sum(w)
    v_s[...] = v_s[...] * scale + jnp.sum(h * w, axis=0, keepdims=True)
    m_s[...] = jnp.full_like(m_s, m_new)

    @pl.when(i == nb - 1)
    def _():
        emb = v_s[...] / s_s[0, 0]
        emb_ref[...] = emb
        c = jnp.dot(_silu(jnp.dot(emb, wc1_ref[...],
                                  preferred_element_type=jnp.float32) + bc1_ref[...]),
                    wc2_ref[...], preferred_element_type=jnp.float32) + bc2_ref[...]
        logits_ref[...] = c
        p = jnp.dot(_silu(jnp.dot(emb, wp1_ref[...],
                                  preferred_element_type=jnp.float32) + bp1_ref[...]),
                    wp2_ref[...], preferred_element_type=jnp.float32) + bp2_ref[...]
        nrm = jnp.maximum(jnp.sqrt(jnp.sum(p * p)), 1e-12)
        proj_ref[...] = p / nrm


def _readout(h, Wg1, bg1, Wg2, bg2, Wc1, bc1, Wc2, bc2, Wp1, bp1, Wp2, bp2):
    N = h.shape[0]
    BN = 1000
    grid = (N // BN,)
    Hq = Wg1.shape[1]
    Hc = Wc1.shape[1]
    C = Wc2.shape[1]
    P = Wp2.shape[1]
    full = lambda r, c: pl.BlockSpec((r, c), lambda i: (0, 0))
    return pl.pallas_call(
        _readout_body,
        grid=grid,
        in_specs=[
            pl.BlockSpec((BN, H), lambda i: (i, 0)),
            full(H, Hq), full(1, Hq), full(Hq, 1), full(1, 1),
            full(H, Hc), full(1, Hc), full(Hc, C), full(1, C),
            full(H, H), full(1, H), full(H, P), full(1, P),
        ],
        out_specs=[full(1, C), full(1, P), full(1, H)],
        out_shape=[
            jax.ShapeDtypeStruct((1, C), jnp.float32),
            jax.ShapeDtypeStruct((1, P), jnp.float32),
            jax.ShapeDtypeStruct((1, H), jnp.float32),
        ],
        scratch_shapes=[
            pltpu.VMEM((1, 128), jnp.float32),
            pltpu.VMEM((1, 128), jnp.float32),
            pltpu.VMEM((1, H), jnp.float32),
        ],
    )(h, Wg1, bg1.reshape(1, Hq), Wg2, bg2.reshape(1, 1),
      Wc1, bc1.reshape(1, Hc), Wc2, bc2.reshape(1, C),
      Wp1, bp1.reshape(1, H), Wp2, bp2.reshape(1, P))


def _geo_aggregate(pos, src, dst, N):
    rel = pos[dst] - pos[src]
    r2 = jnp.sum(rel * rel, axis=-1, keepdims=True)
    d = jnp.sqrt(r2)
    cutoff, R = 10.0, 16
    centers = jnp.linspace(0.0, cutoff, R)
    width = 0.5 * (cutoff / R) ** 2
    rad = jnp.exp(-width * (d - centers[None, :]) ** 2)
    sh = rel / jnp.maximum(d, 1e-12)
    ones = jnp.ones((src.shape[0], 1), jnp.float32)
    zeros = jnp.zeros((src.shape[0], GEOF - 20), jnp.float32)
    feats = jnp.concatenate([sh, rad, ones, zeros], axis=-1)
    return jnp.zeros((N, GEOF), jnp.float32).at[dst].add(feats)


def _segsum(h, src, dst, N):
    E = src.shape[0]
    NC, NS = 2, 16
    EPC = E // NC
    EPT = EPC // NS
    C = 200
    NCHUNK = EPT // C
    NRC = N // C
    MAXT = -(-NRC // NS)
    mesh = plsc.VectorSubcoreMesh(core_axis_name="c", subcore_axis_name="s")

    @functools.partial(
        pl.kernel,
        out_type=jax.ShapeDtypeStruct((NC, N, H), jnp.float32),
        mesh=mesh,
        scratch_types=[
            pltpu.VMEM_SHARED((N, H), jnp.float32),
            pltpu.VMEM((C,), jnp.int32),
            pltpu.VMEM((C,), jnp.int32),
            pltpu.VMEM((C, H), jnp.float32),
            pltpu.SemaphoreType.DMA,
        ],
    )
    def k(h_hbm, src_hbm, dst_hbm, out_hbm, acc, src_v, dst_v, rows_v, sem):
        c = lax.axis_index("c")
        s = lax.axis_index("s")

        zero16 = jnp.zeros((16,), jnp.float32)

        def zbody(r, _):
            for j in range(H // 16):
                rows_v[r, pl.ds(j * 16, 16)] = zero16
            return 0

        lax.fori_loop(0, C, zbody, 0)
        for t in range(MAXT):
            g = s + t * NS

            @pl.when(g < NRC)
            def _():
                pltpu.sync_copy(rows_v, acc.at[pl.ds(g * C, C)])

        plsc.subcore_barrier()

        base0 = c * EPC + s * EPT

        def ebody(kk, _):
            base = base0 + kk * C
            pltpu.sync_copy(src_hbm.at[pl.ds(base, C)], src_v)
            pltpu.sync_copy(dst_hbm.at[pl.ds(base, C)], dst_v)
            pltpu.async_copy(h_hbm.at[src_v], rows_v, sem).wait()
            pltpu.sync_copy(rows_v, acc.at[dst_v], add=True)
            return 0

        lax.fori_loop(0, NCHUNK, ebody, 0)
        plsc.subcore_barrier()

        for t in range(MAXT):
            g = s + t * NS

            @pl.when(g < NRC)
            def _():
                pltpu.sync_copy(acc.at[pl.ds(g * C, C)], rows_v)
                pltpu.sync_copy(rows_v, out_hbm.at[c].at[pl.ds(g * C, C)])

    return k(h, src, dst)


def kernel(x, pos, edge_index, W1, b1, W2, b2, ne_ln_g, ne_ln_b, convW, convB,
           conv_ln_g, conv_ln_b, ln_g, ln_b, Wg1, bg1, Wg2, bg2, Wc1, bc1,
           Wc2, bc2, Wp1, bp1, Wp2, bp2):
    N = x.shape[0]
    L = convW.shape[0]
    src = edge_index[0]
    dst = edge_index[1]

    Wgeo = jnp.concatenate(
        [convW[:, H:H + 19, :], convB[:, None, :],
         jnp.zeros((L, GEOF - 20, H), jnp.float32)], axis=1)
    Wh = convW[:, :H, :]

    h = _encoder(x, W1, b1, W2, b2, ne_ln_g, ne_ln_b)
    geo = _geo_aggregate(pos, src, dst, N)

    for i in range(L):
        seg2 = _segsum(h, src, dst, N)
        h = _layer_dense(seg2, geo, h, Wh[i], Wgeo[i], conv_ln_g[i],
                         conv_ln_b[i], ln_g[i], ln_b[i])

    return _readout(h, Wg1, bg1, Wg2, bg2, Wc1, bc1, Wc2, bc2,
                    Wp1, bp1, Wp2, bp2)


# speedup vs baseline: 5.0592x; 2.2234x over previous
import dataclasses
import functools

import jax
import jax.numpy as jnp
from jax import lax
from jax.experimental import pallas as pl
from jax.experimental.pallas import tpu as pltpu
from jax.experimental.pallas import tpu_sc as plsc

H = 128
GEOF = 128


def _sc_params():
    cp = pltpu.CompilerParams()
    if "needs_layout_passes" in pltpu.CompilerParams.__dataclass_fields__:
        cp = dataclasses.replace(cp, needs_layout_passes=False)
    return cp


def _silu(x):
    return x * jax.nn.sigmoid(x)


def _ln(x, g, b, eps=1e-5):
    m = jnp.mean(x, axis=-1, keepdims=True)
    v = jnp.mean((x - m) ** 2, axis=-1, keepdims=True)
    return (x - m) * jax.lax.rsqrt(v + eps) * g + b


def _enc_body(x_ref, w1_ref, b1_ref, w2_ref, b2_ref, g_ref, bb_ref, o_ref):
    h = jnp.dot(x_ref[...], w1_ref[...], preferred_element_type=jnp.float32)
    h = _silu(h + b1_ref[...])
    h = jnp.dot(h, w2_ref[...], preferred_element_type=jnp.float32) + b2_ref[...]
    o_ref[...] = _ln(h, g_ref[...], bb_ref[...])


def _encoder(x, W1, b1, W2, b2, g, b):
    N, D = x.shape
    BN = 1000
    grid = (N // BN,)
    return pl.pallas_call(
        _enc_body,
        grid=grid,
        in_specs=[
            pl.BlockSpec((BN, D), lambda i: (i, 0)),
            pl.BlockSpec((D, H), lambda i: (0, 0)),
            pl.BlockSpec((1, H), lambda i: (0, 0)),
            pl.BlockSpec((H, H), lambda i: (0, 0)),
            pl.BlockSpec((1, H), lambda i: (0, 0)),
            pl.BlockSpec((1, H), lambda i: (0, 0)),
            pl.BlockSpec((1, H), lambda i: (0, 0)),
        ],
        out_specs=pl.BlockSpec((BN, H), lambda i: (i, 0)),
        out_shape=jax.ShapeDtypeStruct((N, H), jnp.float32),
    )(x, W1, b1.reshape(1, H), W2, b2.reshape(1, H), g.reshape(1, H), b.reshape(1, H))


def _layer_body(seg_ref, geo_ref, h_ref, wh_ref, wg_ref, cg_ref, cb_ref,
                lg_ref, lb_ref, o_ref):
    seg = seg_ref[0] + seg_ref[1]
    agg = jnp.dot(seg, wh_ref[...], preferred_element_type=jnp.float32)
    agg = agg + jnp.dot(geo_ref[...], wg_ref[...], preferred_element_type=jnp.float32)
    h_new = _ln(_silu(agg), cg_ref[...], cb_ref[...])
    o_ref[...] = _ln(h_ref[...] + h_new, lg_ref[...], lb_ref[...])


def _layer_dense(seg2, geo, h, Wh, Wgeo, cg, cb, lg, lb):
    N = h.shape[0]
    BN = 1000
    grid = (N // BN,)
    return pl.pallas_call(
        _layer_body,
        grid=grid,
        in_specs=[
            pl.BlockSpec((2, BN, H), lambda i: (0, i, 0)),
            pl.BlockSpec((BN, GEOF), lambda i: (i, 0)),
            pl.BlockSpec((BN, H), lambda i: (i, 0)),
            pl.BlockSpec((H, H), lambda i: (0, 0)),
            pl.BlockSpec((GEOF, H), lambda i: (0, 0)),
            pl.BlockSpec((1, H), lambda i: (0, 0)),
            pl.BlockSpec((1, H), lambda i: (0, 0)),
            pl.BlockSpec((1, H), lambda i: (0, 0)),
            pl.BlockSpec((1, H), lambda i: (0, 0)),
        ],
        out_specs=pl.BlockSpec((BN, H), lambda i: (i, 0)),
        out_shape=jax.ShapeDtypeStruct((N, H), jnp.float32),
    )(seg2, geo, h, Wh, Wgeo, cg.reshape(1, H), cb.reshape(1, H),
      lg.reshape(1, H), lb.reshape(1, H))


def _readout_body(h_ref, wg1_ref, bg1_ref, wg2_ref, bg2_ref,
                  wc1_ref, bc1_ref, wc2_ref, bc2_ref,
                  wp1_ref, bp1_ref, wp2_ref, bp2_ref,
                  logits_ref, proj_ref, emb_ref,
                  m_s, s_s, v_s):
    i = pl.program_id(0)
    nb = pl.num_programs(0)
    h = h_ref[...]
    z = jnp.dot(_silu(jnp.dot(h, wg1_ref[...],
                              preferred_element_type=jnp.float32) + bg1_ref[...]),
                wg2_ref[...], preferred_element_type=jnp.float32) + bg2_ref[0, 0]
    bm = jnp.max(z)

    @pl.when(i == 0)
    def _():
        m_s[...] = jnp.full_like(m_s, -jnp.inf)
        s_s[...] = jnp.zeros_like(s_s)
        v_s[...] = jnp.zeros_like(v_s)

    m_old = m_s[0, 0]
    m_new = jnp.maximum(m_old, bm)
    scale = jnp.exp(m_old - m_new)
    w = jnp.exp(z - m_new)
    s_s[...] = s_s[...] * scale + jnp.sum(w)
    v_s[...] = v_s[...] * scale + jnp.sum(h * w, axis=0, keepdims=True)
    m_s[...] = jnp.full_like(m_s, m_new)

    @pl.when(i == nb - 1)
    def _():
        emb = v_s[...] / s_s[0, 0]
        emb_ref[...] = emb
        c = jnp.dot(_silu(jnp.dot(emb, wc1_ref[...],
                                  preferred_element_type=jnp.float32) + bc1_ref[...]),
                    wc2_ref[...], preferred_element_type=jnp.float32) + bc2_ref[...]
        logits_ref[...] = c
        p = jnp.dot(_silu(jnp.dot(emb, wp1_ref[...],
                                  preferred_element_type=jnp.float32) + bp1_ref[...]),
                    wp2_ref[...], preferred_element_type=jnp.float32) + bp2_ref[...]
        nrm = jnp.maximum(jnp.sqrt(jnp.sum(p * p)), 1e-12)
        proj_ref[...] = p / nrm


def _readout(h, Wg1, bg1, Wg2, bg2, Wc1, bc1, Wc2, bc2, Wp1, bp1, Wp2, bp2):
    N = h.shape[0]
    BN = 1000
    grid = (N // BN,)
    Hq = Wg1.shape[1]
    Hc = Wc1.shape[1]
    C = Wc2.shape[1]
    P = Wp2.shape[1]
    full = lambda r, c: pl.BlockSpec((r, c), lambda i: (0, 0))
    return pl.pallas_call(
        _readout_body,
        grid=grid,
        in_specs=[
            pl.BlockSpec((BN, H), lambda i: (i, 0)),
            full(H, Hq), full(1, Hq), full(Hq, 1), full(1, 1),
            full(H, Hc), full(1, Hc), full(Hc, C), full(1, C),
            full(H, H), full(1, H), full(H, P), full(1, P),
        ],
        out_specs=[full(1, C), full(1, P), full(1, H)],
        out_shape=[
            jax.ShapeDtypeStruct((1, C), jnp.float32),
            jax.ShapeDtypeStruct((1, P), jnp.float32),
            jax.ShapeDtypeStruct((1, H), jnp.float32),
        ],
        scratch_shapes=[
            pltpu.VMEM((1, 128), jnp.float32),
            pltpu.VMEM((1, 128), jnp.float32),
            pltpu.VMEM((1, H), jnp.float32),
        ],
    )(h, Wg1, bg1.reshape(1, Hq), Wg2, bg2.reshape(1, 1),
      Wc1, bc1.reshape(1, Hc), Wc2, bc2.reshape(1, C),
      Wp1, bp1.reshape(1, H), Wp2, bp2.reshape(1, P))


def _rsqrt_nr(r2):
    bits = plsc.bitcast(r2, jnp.int32)
    y = plsc.bitcast(jnp.int32(0x5F3759DF) - (bits >> 1), jnp.float32)
    for _ in range(4):
        y = y * (1.5 - 0.5 * r2 * y * y)
    return y


def _geo_aggregate(pos, src, dst, N):
    E = src.shape[0]
    NC, NS, LN = 2, 16, 16
    C = 320
    NCH = E // C
    MAXT = -(-NCH // NS)
    NR = N // NC
    NRP = NR + 8
    RC = 200
    NRC = NR // RC
    cutoff, R = 10.0, 16
    width = 0.5 * (cutoff / R) ** 2
    centers = [cutoff * kk / (R - 1) for kk in range(R)]
    mesh = plsc.VectorSubcoreMesh(core_axis_name="c", subcore_axis_name="s", num_cores=2, num_subcores=16)

    @functools.partial(
        pl.kernel,
        out_type=jax.ShapeDtypeStruct((N, GEOF), jnp.float32),
        mesh=mesh,
        compiler_params=_sc_params(),
        scratch_types=[
            pltpu.VMEM_SHARED((NRP, GEOF), jnp.float32),
            pltpu.VMEM((N,), jnp.float32),
            pltpu.VMEM((N,), jnp.float32),
            pltpu.VMEM((N,), jnp.float32),
            pltpu.VMEM((C,), jnp.int32),
            pltpu.VMEM((C,), jnp.int32),
            pltpu.VMEM((C,), jnp.int32),
            pltpu.VMEM((C, GEOF), jnp.float32),
        ],
    )
    def k(px_hbm, py_hbm, pz_hbm, src_hbm, dst_hbm, out_hbm, gacc,
          px_v, py_v, pz_v, src_v, dst_v, loc_v, fbuf):
        c = lax.axis_index("c")
        s = lax.axis_index("s")
        zero16 = jnp.zeros((LN,), jnp.float32)
        lo = c * NR

        pltpu.sync_copy(px_hbm, px_v)
        pltpu.sync_copy(py_hbm, py_v)
        pltpu.sync_copy(pz_hbm, pz_v)

        def zbody(r, _):
            for j in range(GEOF // LN):
                fbuf[r, pl.ds(j * LN, LN)] = zero16
            return 0

        lax.fori_loop(0, C, zbody, 0)
        for t in range(-(-NRC // NS)):
            g = s + t * NS

            @pl.when(g < NRC)
            def _():
                pltpu.sync_copy(fbuf.at[pl.ds(0, RC)], gacc.at[pl.ds(g * RC, RC)])

        plsc.subcore_barrier()

        iota = lax.iota(jnp.int32, LN)
        fcols = [jnp.full((LN,), f, jnp.int32) for f in range(20)]

        def chunk(g):
            base = g * C
            pltpu.sync_copy(src_hbm.at[pl.ds(base, C)], src_v)
            pltpu.sync_copy(dst_hbm.at[pl.ds(base, C)], dst_v)

            def grp(j, _):
                rows = j * LN + iota
                s16 = src_v[pl.ds(j * LN, LN)]
                d16 = dst_v[pl.ds(j * LN, LN)]
                dloc = d16 - lo
                inrange = (dloc >= 0) & (dloc < NR)
                loc_v[pl.ds(j * LN, LN)] = jnp.where(inrange, dloc, NR)
                relx = plsc.load_gather(px_v, [d16]) - plsc.load_gather(px_v, [s16])
                rely = plsc.load_gather(py_v, [d16]) - plsc.load_gather(py_v, [s16])
                relz = plsc.load_gather(pz_v, [d16]) - plsc.load_gather(pz_v, [s16])
                r2 = relx * relx + rely * rely + relz * relz
                y = _rsqrt_nr(r2)
                d = r2 * y
                inv = jnp.minimum(y, 1e12)
                plsc.store_scatter(fbuf, [rows, fcols[0]], relx * inv)
                plsc.store_scatter(fbuf, [rows, fcols[1]], rely * inv)
                plsc.store_scatter(fbuf, [rows, fcols[2]], relz * inv)
                for kk in range(R):
                    dk = d - centers[kk]
                    plsc.store_scatter(fbuf, [rows, fcols[3 + kk]],
                                       jnp.exp(-width * dk * dk))
                plsc.store_scatter(fbuf, [rows, fcols[19]],
                                   zero16 + 1.0)
                return 0

            lax.fori_loop(0, C // LN, grp, 0)
            pltpu.sync_copy(fbuf, gacc.at[loc_v], add=True)

        for t in range(MAXT):
            g = s + t * NS

            @pl.when(g < NCH)
            def _():
                chunk(g)

        plsc.subcore_barrier()

        for t in range(-(-NRC // NS)):
            g = s + t * NS

            @pl.when(g < NRC)
            def _():
                pltpu.sync_copy(gacc.at[pl.ds(g * RC, RC)], fbuf.at[pl.ds(0, RC)])
                pltpu.sync_copy(fbuf.at[pl.ds(0, RC)],
                                out_hbm.at[pl.ds(lo + g * RC, RC)])

    return k(pos[:, 0].astype(jnp.float32), pos[:, 1].astype(jnp.float32),
             pos[:, 2].astype(jnp.float32), src, dst)


def _segsum(h, src, dst, N):
    E = src.shape[0]
    NC, NS = 2, 16
    EPC = E // NC
    EPT = EPC // NS
    C = 200
    NCHUNK = EPT // C
    NRC = N // C
    MAXT = -(-NRC // NS)
    mesh = plsc.VectorSubcoreMesh(core_axis_name="c", subcore_axis_name="s", num_cores=2, num_subcores=16)

    @functools.partial(
        pl.kernel,
        out_type=jax.ShapeDtypeStruct((NC, N, H), jnp.float32),
        mesh=mesh,
        scratch_types=[
            pltpu.VMEM_SHARED((N, H), jnp.float32),
            pltpu.VMEM((C,), jnp.int32),
            pltpu.VMEM((C,), jnp.int32),
            pltpu.VMEM((C, H), jnp.float32),
            pltpu.SemaphoreType.DMA,
        ],
    )
    def k(h_hbm, src_hbm, dst_hbm, out_hbm, acc, src_v, dst_v, rows_v, sem):
        c = lax.axis_index("c")
        s = lax.axis_index("s")

        zero16 = jnp.zeros((16,), jnp.float32)

        def zbody(r, _):
            for j in range(H // 16):
                rows_v[r, pl.ds(j * 16, 16)] = zero16
            return 0

        lax.fori_loop(0, C, zbody, 0)
        for t in range(MAXT):
            g = s + t * NS

            @pl.when(g < NRC)
            def _():
                pltpu.sync_copy(rows_v, acc.at[pl.ds(g * C, C)])

        plsc.subcore_barrier()

        base0 = c * EPC + s * EPT

        def ebody(kk, _):
            base = base0 + kk * C
            pltpu.sync_copy(src_hbm.at[pl.ds(base, C)], src_v)
            pltpu.sync_copy(dst_hbm.at[pl.ds(base, C)], dst_v)
            pltpu.async_copy(h_hbm.at[src_v], rows_v, sem).wait()
            pltpu.sync_copy(rows_v, acc.at[dst_v], add=True)
            return 0

        lax.fori_loop(0, NCHUNK, ebody, 0)
        plsc.subcore_barrier()

        for t in range(MAXT):
            g = s + t * NS

            @pl.when(g < NRC)
            def _():
                pltpu.sync_copy(acc.at[pl.ds(g * C, C)], rows_v)
                pltpu.sync_copy(rows_v, out_hbm.at[c].at[pl.ds(g * C, C)])

    return k(h, src, dst)


def kernel(x, pos, edge_index, W1, b1, W2, b2, ne_ln_g, ne_ln_b, convW, convB,
           conv_ln_g, conv_ln_b, ln_g, ln_b, Wg1, bg1, Wg2, bg2, Wc1, bc1,
           Wc2, bc2, Wp1, bp1, Wp2, bp2):
    N = x.shape[0]
    L = convW.shape[0]
    src = edge_index[0]
    dst = edge_index[1]

    Wgeo = jnp.concatenate(
        [convW[:, H:H + 19, :], convB[:, None, :],
         jnp.zeros((L, GEOF - 20, H), jnp.float32)], axis=1)
    Wh = convW[:, :H, :]

    h = _encoder(x, W1, b1, W2, b2, ne_ln_g, ne_ln_b)
    geo = _geo_aggregate(pos, src, dst, N)

    for i in range(L):
        seg2 = _segsum(h, src, dst, N)
        h = _layer_dense(seg2, geo, h, Wh[i], Wgeo[i], conv_ln_g[i],
                         conv_ln_b[i], ln_g[i], ln_b[i])

    return _readout(h, Wg1, bg1, Wg2, bg2, Wc1, bc1, Wc2, bc2,
                    Wp1, bp1, Wp2, bp2)


# speedup vs baseline: 5.8066x; 1.1477x over previous
import dataclasses
import functools

import jax
import jax.numpy as jnp
from jax import lax
from jax.experimental import pallas as pl
from jax.experimental.pallas import tpu as pltpu
from jax.experimental.pallas import tpu_sc as plsc

H = 128
GEOF = 128


def _sc_params():
    cp = pltpu.CompilerParams()
    if "needs_layout_passes" in pltpu.CompilerParams.__dataclass_fields__:
        cp = dataclasses.replace(cp, needs_layout_passes=False)
    return cp


def _silu(x):
    return x * jax.nn.sigmoid(x)


def _ln(x, g, b, eps=1e-5):
    m = jnp.mean(x, axis=-1, keepdims=True)
    v = jnp.mean((x - m) ** 2, axis=-1, keepdims=True)
    return (x - m) * jax.lax.rsqrt(v + eps) * g + b


def _enc_body(x_ref, w1_ref, b1_ref, w2_ref, b2_ref, g_ref, bb_ref, o_ref):
    h = jnp.dot(x_ref[...], w1_ref[...], preferred_element_type=jnp.float32)
    h = _silu(h + b1_ref[...])
    h = jnp.dot(h, w2_ref[...], preferred_element_type=jnp.float32) + b2_ref[...]
    o_ref[...] = _ln(h, g_ref[...], bb_ref[...])


def _encoder(x, W1, b1, W2, b2, g, b):
    N, D = x.shape
    BN = 1000
    grid = (N // BN,)
    return pl.pallas_call(
        _enc_body,
        grid=grid,
        in_specs=[
            pl.BlockSpec((BN, D), lambda i: (i, 0)),
            pl.BlockSpec((D, H), lambda i: (0, 0)),
            pl.BlockSpec((1, H), lambda i: (0, 0)),
            pl.BlockSpec((H, H), lambda i: (0, 0)),
            pl.BlockSpec((1, H), lambda i: (0, 0)),
            pl.BlockSpec((1, H), lambda i: (0, 0)),
            pl.BlockSpec((1, H), lambda i: (0, 0)),
        ],
        out_specs=pl.BlockSpec((BN, H), lambda i: (i, 0)),
        out_shape=jax.ShapeDtypeStruct((N, H), jnp.float32),
    )(x, W1, b1.reshape(1, H), W2, b2.reshape(1, H), g.reshape(1, H), b.reshape(1, H))


def _layer_body(seg_ref, geo_ref, h_ref, wh_ref, wg_ref, cg_ref, cb_ref,
                lg_ref, lb_ref, o_ref):
    seg = seg_ref[0] + seg_ref[1]
    agg = jnp.dot(seg, wh_ref[...], preferred_element_type=jnp.float32)
    agg = agg + jnp.dot(geo_ref[...], wg_ref[...], preferred_element_type=jnp.float32)
    h_new = _ln(_silu(agg), cg_ref[...], cb_ref[...])
    o_ref[...] = _ln(h_ref[...] + h_new, lg_ref[...], lb_ref[...])


def _layer_dense(seg2, geo, h, Wh, Wgeo, cg, cb, lg, lb):
    N = h.shape[0]
    BN = 1000
    grid = (N // BN,)
    return pl.pallas_call(
        _layer_body,
        grid=grid,
        in_specs=[
            pl.BlockSpec((2, BN, H), lambda i: (0, i, 0)),
            pl.BlockSpec((BN, GEOF), lambda i: (i, 0)),
            pl.BlockSpec((BN, H), lambda i: (i, 0)),
            pl.BlockSpec((H, H), lambda i: (0, 0)),
            pl.BlockSpec((GEOF, H), lambda i: (0, 0)),
            pl.BlockSpec((1, H), lambda i: (0, 0)),
            pl.BlockSpec((1, H), lambda i: (0, 0)),
            pl.BlockSpec((1, H), lambda i: (0, 0)),
            pl.BlockSpec((1, H), lambda i: (0, 0)),
        ],
        out_specs=pl.BlockSpec((BN, H), lambda i: (i, 0)),
        out_shape=jax.ShapeDtypeStruct((N, H), jnp.float32),
    )(seg2, geo, h, Wh, Wgeo, cg.reshape(1, H), cb.reshape(1, H),
      lg.reshape(1, H), lb.reshape(1, H))


def _readout_body(h_ref, wg1_ref, bg1_ref, wg2_ref, bg2_ref,
                  wc1_ref, bc1_ref, wc2_ref, bc2_ref,
                  wp1_ref, bp1_ref, wp2_ref, bp2_ref,
                  logits_ref, proj_ref, emb_ref,
                  m_s, s_s, v_s):
    i = pl.program_id(0)
    nb = pl.num_programs(0)
    h = h_ref[...]
    z = jnp.dot(_silu(jnp.dot(h, wg1_ref[...],
                              preferred_element_type=jnp.float32) + bg1_ref[...]),
                wg2_ref[...], preferred_element_type=jnp.float32) + bg2_ref[0, 0]
    bm = jnp.max(z)

    @pl.when(i == 0)
    def _():
        m_s[...] = jnp.full_like(m_s, -jnp.inf)
        s_s[...] = jnp.zeros_like(s_s)
        v_s[...] = jnp.zeros_like(v_s)

    m_old = m_s[0, 0]
    m_new = jnp.maximum(m_old, bm)
    scale = jnp.exp(m_old - m_new)
    w = jnp.exp(z - m_new)
    s_s[...] = s_s[...] * scale + jnp.sum(w)
    v_s[...] = v_s[...] * scale + jnp.sum(h * w, axis=0, keepdims=True)
    m_s[...] = jnp.full_like(m_s, m_new)

    @pl.when(i == nb - 1)
    def _():
        emb = v_s[...] / s_s[0, 0]
        emb_ref[...] = emb
        c = jnp.dot(_silu(jnp.dot(emb, wc1_ref[...],
                                  preferred_element_type=jnp.float32) + bc1_ref[...]),
                    wc2_ref[...], preferred_element_type=jnp.float32) + bc2_ref[...]
        logits_ref[...] = c
        p = jnp.dot(_silu(jnp.dot(emb, wp1_ref[...],
                                  preferred_element_type=jnp.float32) + bp1_ref[...]),
                    wp2_ref[...], preferred_element_type=jnp.float32) + bp2_ref[...]
        nrm = jnp.maximum(jnp.sqrt(jnp.sum(p * p)), 1e-12)
        proj_ref[...] = p / nrm


def _readout(h, Wg1, bg1, Wg2, bg2, Wc1, bc1, Wc2, bc2, Wp1, bp1, Wp2, bp2):
    N = h.shape[0]
    BN = 1000
    grid = (N // BN,)
    Hq = Wg1.shape[1]
    Hc = Wc1.shape[1]
    C = Wc2.shape[1]
    P = Wp2.shape[1]
    full = lambda r, c: pl.BlockSpec((r, c), lambda i: (0, 0))
    return pl.pallas_call(
        _readout_body,
        grid=grid,
        in_specs=[
            pl.BlockSpec((BN, H), lambda i: (i, 0)),
            full(H, Hq), full(1, Hq), full(Hq, 1), full(1, 1),
            full(H, Hc), full(1, Hc), full(Hc, C), full(1, C),
            full(H, H), full(1, H), full(H, P), full(1, P),
        ],
        out_specs=[full(1, C), full(1, P), full(1, H)],
        out_shape=[
            jax.ShapeDtypeStruct((1, C), jnp.float32),
            jax.ShapeDtypeStruct((1, P), jnp.float32),
            jax.ShapeDtypeStruct((1, H), jnp.float32),
        ],
        scratch_shapes=[
            pltpu.VMEM((1, 128), jnp.float32),
            pltpu.VMEM((1, 128), jnp.float32),
            pltpu.VMEM((1, H), jnp.float32),
        ],
    )(h, Wg1, bg1.reshape(1, Hq), Wg2, bg2.reshape(1, 1),
      Wc1, bc1.reshape(1, Hc), Wc2, bc2.reshape(1, C),
      Wp1, bp1.reshape(1, H), Wp2, bp2.reshape(1, P))


def _rsqrt_nr(r2):
    bits = plsc.bitcast(r2, jnp.int32)
    y = plsc.bitcast(jnp.int32(0x5F3759DF) - (bits >> 1), jnp.float32)
    for _ in range(4):
        y = y * (1.5 - 0.5 * r2 * y * y)
    return y


def _geo_aggregate(pos, src, dst, N):
    E = src.shape[0]
    NC, NS, LN = 2, 16, 16
    C = 320
    NCH = E // C
    MAXT = -(-NCH // NS)
    NR = N // NC
    NRP = NR + 8
    RC = 200
    NRC = NR // RC
    cutoff, R = 10.0, 16
    width = 0.5 * (cutoff / R) ** 2
    centers = [cutoff * kk / (R - 1) for kk in range(R)]
    mesh = plsc.VectorSubcoreMesh(core_axis_name="c", subcore_axis_name="s", num_cores=2, num_subcores=16)

    @functools.partial(
        pl.kernel,
        out_type=jax.ShapeDtypeStruct((N, GEOF), jnp.float32),
        mesh=mesh,
        compiler_params=_sc_params(),
        scratch_types=[
            pltpu.VMEM_SHARED((NRP, GEOF), jnp.float32),
            pltpu.VMEM((N,), jnp.float32),
            pltpu.VMEM((N,), jnp.float32),
            pltpu.VMEM((N,), jnp.float32),
            pltpu.VMEM((C,), jnp.int32),
            pltpu.VMEM((C,), jnp.int32),
            pltpu.VMEM((C,), jnp.int32),
            pltpu.VMEM((C, GEOF), jnp.float32),
        ],
    )
    def k(px_hbm, py_hbm, pz_hbm, src_hbm, dst_hbm, out_hbm, gacc,
          px_v, py_v, pz_v, src_v, dst_v, loc_v, fbuf):
        c = lax.axis_index("c")
        s = lax.axis_index("s")
        zero16 = jnp.zeros((LN,), jnp.float32)
        lo = c * NR

        pltpu.sync_copy(px_hbm, px_v)
        pltpu.sync_copy(py_hbm, py_v)
        pltpu.sync_copy(pz_hbm, pz_v)

        def zbody(r, _):
            for j in range(GEOF // LN):
                fbuf[r, pl.ds(j * LN, LN)] = zero16
            return 0

        lax.fori_loop(0, C, zbody, 0)
        for t in range(-(-NRC // NS)):
            g = s + t * NS

            @pl.when(g < NRC)
            def _():
                pltpu.sync_copy(fbuf.at[pl.ds(0, RC)], gacc.at[pl.ds(g * RC, RC)])

        plsc.subcore_barrier()

        iota = lax.iota(jnp.int32, LN)
        fcols = [jnp.full((LN,), f, jnp.int32) for f in range(20)]

        def chunk(g):
            base = g * C
            pltpu.sync_copy(src_hbm.at[pl.ds(base, C)], src_v)
            pltpu.sync_copy(dst_hbm.at[pl.ds(base, C)], dst_v)

            def grp(j, _):
                rows = j * LN + iota
                s16 = src_v[pl.ds(j * LN, LN)]
                d16 = dst_v[pl.ds(j * LN, LN)]
                dloc = d16 - lo
                inrange = (dloc >= 0) & (dloc < NR)
                loc_v[pl.ds(j * LN, LN)] = jnp.where(inrange, dloc, NR)
                relx = plsc.load_gather(px_v, [d16]) - plsc.load_gather(px_v, [s16])
                rely = plsc.load_gather(py_v, [d16]) - plsc.load_gather(py_v, [s16])
                relz = plsc.load_gather(pz_v, [d16]) - plsc.load_gather(pz_v, [s16])
                r2 = relx * relx + rely * rely + relz * relz
                y = _rsqrt_nr(r2)
                d = r2 * y
                inv = jnp.minimum(y, 1e12)
                plsc.store_scatter(fbuf, [rows, fcols[0]], relx * inv)
                plsc.store_scatter(fbuf, [rows, fcols[1]], rely * inv)
                plsc.store_scatter(fbuf, [rows, fcols[2]], relz * inv)
                for kk in range(R):
                    dk = d - centers[kk]
                    plsc.store_scatter(fbuf, [rows, fcols[3 + kk]],
                                       jnp.exp(-width * dk * dk))
                plsc.store_scatter(fbuf, [rows, fcols[19]],
                                   zero16 + 1.0)
                return 0

            lax.fori_loop(0, C // LN, grp, 0)
            pltpu.sync_copy(fbuf, gacc.at[loc_v], add=True)

        for t in range(MAXT):
            g = s + t * NS

            @pl.when(g < NCH)
            def _():
                chunk(g)

        plsc.subcore_barrier()

        for t in range(-(-NRC // NS)):
            g = s + t * NS

            @pl.when(g < NRC)
            def _():
                pltpu.sync_copy(gacc.at[pl.ds(g * RC, RC)], fbuf.at[pl.ds(0, RC)])
                pltpu.sync_copy(fbuf.at[pl.ds(0, RC)],
                                out_hbm.at[pl.ds(lo + g * RC, RC)])

    return k(pos[:, 0].astype(jnp.float32), pos[:, 1].astype(jnp.float32),
             pos[:, 2].astype(jnp.float32), src, dst)


def _segsum(h, src, dst, N):
    E = src.shape[0]
    NC, NS = 2, 16
    EPC = E // NC
    C = 128
    NCH = EPC // C
    NPAIR = -(-NCH // NS) // 2 + 1
    RC = 80
    NRC = N // RC
    mesh = plsc.VectorSubcoreMesh(core_axis_name="c", subcore_axis_name="s", num_cores=2, num_subcores=16)

    @functools.partial(
        pl.kernel,
        out_type=jax.ShapeDtypeStruct((NC, N, H), jnp.float32),
        mesh=mesh,
        scratch_types=[
            pltpu.VMEM_SHARED((N, H), jnp.float32),
            pltpu.VMEM((C,), jnp.int32),
            pltpu.VMEM((C,), jnp.int32),
            pltpu.VMEM((C,), jnp.int32),
            pltpu.VMEM((C,), jnp.int32),
            pltpu.VMEM((C, H), jnp.float32),
            pltpu.VMEM((C, H), jnp.float32),
            pltpu.SemaphoreType.DMA,
            pltpu.SemaphoreType.DMA,
        ],
    )
    def k(h_hbm, src_hbm, dst_hbm, out_hbm, acc, srcA, dstA, srcB, dstB,
          rowsA, rowsB, semA, semB):
        c = lax.axis_index("c")
        s = lax.axis_index("s")

        zero16 = jnp.zeros((16,), jnp.float32)

        def zbody(r, _):
            for j in range(H // 16):
                rowsA[r, pl.ds(j * 16, 16)] = zero16
            return 0

        lax.fori_loop(0, C, zbody, 0)
        for t in range(-(-NRC // NS)):
            g = s + t * NS

            @pl.when(g < NRC)
            def _():
                pltpu.sync_copy(rowsA.at[pl.ds(0, RC)],
                                acc.at[pl.ds(g * RC, RC)])

        plsc.subcore_barrier()

        e0 = c * EPC

        def load(g, src_v, dst_v):
            base = e0 + g * C
            pltpu.sync_copy(src_hbm.at[pl.ds(base, C)], src_v)
            pltpu.sync_copy(dst_hbm.at[pl.ds(base, C)], dst_v)

        load(s, srcA, dstA)
        pltpu.async_copy(h_hbm.at[srcA], rowsA, semA)

        def pair(u, _):
            g0 = s + (2 * u) * NS
            g1 = s + (2 * u + 1) * NS
            g2 = s + (2 * u + 2) * NS

            @pl.when(g1 < NCH)
            def _():
                load(g1, srcB, dstB)
                pltpu.async_copy(h_hbm.at[srcB], rowsB, semB)

            @pl.when(g0 < NCH)
            def _():
                pltpu.make_async_copy(h_hbm.at[srcA], rowsA, semA).wait()
                pltpu.sync_copy(rowsA, acc.at[dstA], add=True)

            @pl.when(g2 < NCH)
            def _():
                load(g2, srcA, dstA)
                pltpu.async_copy(h_hbm.at[srcA], rowsA, semA)

            @pl.when(g1 < NCH)
            def _():
                pltpu.make_async_copy(h_hbm.at[srcB], rowsB, semB).wait()
                pltpu.sync_copy(rowsB, acc.at[dstB], add=True)

            return 0

        lax.fori_loop(0, NPAIR, pair, 0)
        plsc.subcore_barrier()

        for t in range(-(-NRC // NS)):
            g = s + t * NS

            @pl.when(g < NRC)
            def _():
                pltpu.sync_copy(acc.at[pl.ds(g * RC, RC)],
                                rowsA.at[pl.ds(0, RC)])
                pltpu.sync_copy(rowsA.at[pl.ds(0, RC)],
                                out_hbm.at[c].at[pl.ds(g * RC, RC)])

    return k(h, src, dst)


def kernel(x, pos, edge_index, W1, b1, W2, b2, ne_ln_g, ne_ln_b, convW, convB,
           conv_ln_g, conv_ln_b, ln_g, ln_b, Wg1, bg1, Wg2, bg2, Wc1, bc1,
           Wc2, bc2, Wp1, bp1, Wp2, bp2):
    N = x.shape[0]
    L = convW.shape[0]
    src = edge_index[0]
    dst = edge_index[1]

    Wgeo = jnp.concatenate(
        [convW[:, H:H + 19, :], convB[:, None, :],
         jnp.zeros((L, GEOF - 20, H), jnp.float32)], axis=1)
    Wh = convW[:, :H, :]

    h = _encoder(x, W1, b1, W2, b2, ne_ln_g, ne_ln_b)
    geo = _geo_aggregate(pos, src, dst, N)

    for i in range(L):
        seg2 = _segsum(h, src, dst, N)
        h = _layer_dense(seg2, geo, h, Wh[i], Wgeo[i], conv_ln_g[i],
                         conv_ln_b[i], ln_g[i], ln_b[i])

    return _readout(h, Wg1, bg1, Wg2, bg2, Wc1, bc1, Wc2, bc2,
                    Wp1, bp1, Wp2, bp2)


# speedup vs baseline: 6.2282x; 1.0726x over previous
import dataclasses
import functools

import jax
import jax.numpy as jnp
from jax import lax
from jax.experimental import pallas as pl
from jax.experimental.pallas import tpu as pltpu
from jax.experimental.pallas import tpu_sc as plsc

H = 128
GEOF = 128


def _sc_params():
    cp = pltpu.CompilerParams()
    if "needs_layout_passes" in pltpu.CompilerParams.__dataclass_fields__:
        cp = dataclasses.replace(cp, needs_layout_passes=False)
    return cp


def _silu(x):
    return x * jax.nn.sigmoid(x)


def _ln(x, g, b, eps=1e-5):
    m = jnp.mean(x, axis=-1, keepdims=True)
    v = jnp.mean((x - m) ** 2, axis=-1, keepdims=True)
    return (x - m) * jax.lax.rsqrt(v + eps) * g + b


def _enc_body(x_ref, w1_ref, b1_ref, w2_ref, b2_ref, g_ref, bb_ref, o_ref):
    h = jnp.dot(x_ref[...], w1_ref[...], preferred_element_type=jnp.float32)
    h = _silu(h + b1_ref[...])
    h = jnp.dot(h, w2_ref[...], preferred_element_type=jnp.float32) + b2_ref[...]
    o_ref[...] = _ln(h, g_ref[...], bb_ref[...])


def _encoder(x, W1, b1, W2, b2, g, b):
    N, D = x.shape
    BN = 1000
    grid = (N // BN,)
    return pl.pallas_call(
        _enc_body,
        grid=grid,
        in_specs=[
            pl.BlockSpec((BN, D), lambda i: (i, 0)),
            pl.BlockSpec((D, H), lambda i: (0, 0)),
            pl.BlockSpec((1, H), lambda i: (0, 0)),
            pl.BlockSpec((H, H), lambda i: (0, 0)),
            pl.BlockSpec((1, H), lambda i: (0, 0)),
            pl.BlockSpec((1, H), lambda i: (0, 0)),
            pl.BlockSpec((1, H), lambda i: (0, 0)),
        ],
        out_specs=pl.BlockSpec((BN, H), lambda i: (i, 0)),
        out_shape=jax.ShapeDtypeStruct((N, H), jnp.float32),
    )(x, W1, b1.reshape(1, H), W2, b2.reshape(1, H), g.reshape(1, H), b.reshape(1, H))


def _layer_body(seg_ref, geo_ref, h_ref, wh_ref, wg_ref, cg_ref, cb_ref,
                lg_ref, lb_ref, o_ref):
    seg = seg_ref[0] + seg_ref[1]
    geo = geo_ref[0] + geo_ref[1]
    agg = jnp.dot(seg, wh_ref[...], preferred_element_type=jnp.float32)
    agg = agg + jnp.dot(geo, wg_ref[...], preferred_element_type=jnp.float32)
    h_new = _ln(_silu(agg), cg_ref[...], cb_ref[...])
    o_ref[...] = _ln(h_ref[...] + h_new, lg_ref[...], lb_ref[...])


def _layer_dense(seg2, geo, h, Wh, Wgeo, cg, cb, lg, lb):
    N = h.shape[0]
    BN = 1000
    grid = (N // BN,)
    return pl.pallas_call(
        _layer_body,
        grid=grid,
        in_specs=[
            pl.BlockSpec((2, BN, H), lambda i: (0, i, 0)),
            pl.BlockSpec((2, BN, GEOF), lambda i: (0, i, 0)),
            pl.BlockSpec((BN, H), lambda i: (i, 0)),
            pl.BlockSpec((H, H), lambda i: (0, 0)),
            pl.BlockSpec((GEOF, H), lambda i: (0, 0)),
            pl.BlockSpec((1, H), lambda i: (0, 0)),
            pl.BlockSpec((1, H), lambda i: (0, 0)),
            pl.BlockSpec((1, H), lambda i: (0, 0)),
            pl.BlockSpec((1, H), lambda i: (0, 0)),
        ],
        out_specs=pl.BlockSpec((BN, H), lambda i: (i, 0)),
        out_shape=jax.ShapeDtypeStruct((N, H), jnp.float32),
    )(seg2, geo, h, Wh, Wgeo, cg.reshape(1, H), cb.reshape(1, H),
      lg.reshape(1, H), lb.reshape(1, H))


def _readout_body(h_ref, wg1_ref, bg1_ref, wg2_ref, bg2_ref,
                  wc1_ref, bc1_ref, wc2_ref, bc2_ref,
                  wp1_ref, bp1_ref, wp2_ref, bp2_ref,
                  logits_ref, proj_ref, emb_ref,
                  m_s, s_s, v_s):
    i = pl.program_id(0)
    nb = pl.num_programs(0)
    h = h_ref[...]
    z = jnp.dot(_silu(jnp.dot(h, wg1_ref[...],
                              preferred_element_type=jnp.float32) + bg1_ref[...]),
                wg2_ref[...], preferred_element_type=jnp.float32) + bg2_ref[0, 0]
    bm = jnp.max(z)

    @pl.when(i == 0)
    def _():
        m_s[...] = jnp.full_like(m_s, -jnp.inf)
        s_s[...] = jnp.zeros_like(s_s)
        v_s[...] = jnp.zeros_like(v_s)

    m_old = m_s[0, 0]
    m_new = jnp.maximum(m_old, bm)
    scale = jnp.exp(m_old - m_new)
    w = jnp.exp(z - m_new)
    s_s[...] = s_s[...] * scale + jnp.sum(w)
    v_s[...] = v_s[...] * scale + jnp.sum(h * w, axis=0, keepdims=True)
    m_s[...] = jnp.full_like(m_s, m_new)

    @pl.when(i == nb - 1)
    def _():
        emb = v_s[...] / s_s[0, 0]
        emb_ref[...] = emb
        c = jnp.dot(_silu(jnp.dot(emb, wc1_ref[...],
                                  preferred_element_type=jnp.float32) + bc1_ref[...]),
                    wc2_ref[...], preferred_element_type=jnp.float32) + bc2_ref[...]
        logits_ref[...] = c
        p = jnp.dot(_silu(jnp.dot(emb, wp1_ref[...],
                                  preferred_element_type=jnp.float32) + bp1_ref[...]),
                    wp2_ref[...], preferred_element_type=jnp.float32) + bp2_ref[...]
        nrm = jnp.maximum(jnp.sqrt(jnp.sum(p * p)), 1e-12)
        proj_ref[...] = p / nrm


def _readout(h, Wg1, bg1, Wg2, bg2, Wc1, bc1, Wc2, bc2, Wp1, bp1, Wp2, bp2):
    N = h.shape[0]
    BN = 1000
    grid = (N // BN,)
    Hq = Wg1.shape[1]
    Hc = Wc1.shape[1]
    C = Wc2.shape[1]
    P = Wp2.shape[1]
    full = lambda r, c: pl.BlockSpec((r, c), lambda i: (0, 0))
    return pl.pallas_call(
        _readout_body,
        grid=grid,
        in_specs=[
            pl.BlockSpec((BN, H), lambda i: (i, 0)),
            full(H, Hq), full(1, Hq), full(Hq, 1), full(1, 1),
            full(H, Hc), full(1, Hc), full(Hc, C), full(1, C),
            full(H, H), full(1, H), full(H, P), full(1, P),
        ],
        out_specs=[full(1, C), full(1, P), full(1, H)],
        out_shape=[
            jax.ShapeDtypeStruct((1, C), jnp.float32),
            jax.ShapeDtypeStruct((1, P), jnp.float32),
            jax.ShapeDtypeStruct((1, H), jnp.float32),
        ],
        scratch_shapes=[
            pltpu.VMEM((1, 128), jnp.float32),
            pltpu.VMEM((1, 128), jnp.float32),
            pltpu.VMEM((1, H), jnp.float32),
        ],
    )(h, Wg1, bg1.reshape(1, Hq), Wg2, bg2.reshape(1, 1),
      Wc1, bc1.reshape(1, Hc), Wc2, bc2.reshape(1, C),
      Wp1, bp1.reshape(1, H), Wp2, bp2.reshape(1, P))


def _rsqrt_nr(r2):
    bits = plsc.bitcast(r2, jnp.int32)
    y = plsc.bitcast(jnp.int32(0x5F3759DF) - (bits >> 1), jnp.float32)
    for _ in range(4):
        y = y * (1.5 - 0.5 * r2 * y * y)
    return y


def _geo_aggregate(pos, src, dst, N):
    E = src.shape[0]
    NC, NS, LN = 2, 16, 16
    C = 64
    NCH = E // NC // C
    MAXT = -(-NCH // NS)
    RC = 40
    NRC = N // RC
    cutoff, R = 10.0, 16
    width = 0.5 * (cutoff / R) ** 2
    centers = [cutoff * kk / (R - 1) for kk in range(R)]
    mesh = plsc.VectorSubcoreMesh(core_axis_name="c", subcore_axis_name="s", num_cores=2, num_subcores=16)

    @functools.partial(
        pl.kernel,
        out_type=jax.ShapeDtypeStruct((NC, N, GEOF), jnp.float32),
        mesh=mesh,
        compiler_params=_sc_params(),
        scratch_types=[
            pltpu.VMEM_SHARED((N, GEOF), jnp.float32),
            pltpu.VMEM((N,), jnp.float32),
            pltpu.VMEM((N,), jnp.float32),
            pltpu.VMEM((N,), jnp.float32),
            pltpu.VMEM((C,), jnp.int32),
            pltpu.VMEM((C,), jnp.int32),
            pltpu.VMEM((C, GEOF), jnp.float32),
        ],
    )
    def k(px_hbm, py_hbm, pz_hbm, src_hbm, dst_hbm, out_hbm, gacc,
          px_v, py_v, pz_v, src_v, dst_v, fbuf):
        c = lax.axis_index("c")
        s = lax.axis_index("s")
        zero16 = jnp.zeros((LN,), jnp.float32)

        pltpu.sync_copy(px_hbm, px_v)
        pltpu.sync_copy(py_hbm, py_v)
        pltpu.sync_copy(pz_hbm, pz_v)

        def zbody(r, _):
            for j in range(GEOF // LN):
                fbuf[r, pl.ds(j * LN, LN)] = zero16
            return 0

        lax.fori_loop(0, C, zbody, 0)

        def zrow(t, _):
            g = s + t * NS

            @pl.when(g < NRC)
            def _():
                pltpu.sync_copy(fbuf.at[pl.ds(0, RC)], gacc.at[pl.ds(g * RC, RC)])

            return 0

        lax.fori_loop(0, -(-NRC // NS), zrow, 0)
        plsc.subcore_barrier()

        iota = lax.iota(jnp.int32, LN)
        fcols = [jnp.full((LN,), f, jnp.int32) for f in range(20)]

        def chunk(g):
            base = c * (E // NC) + g * C
            pltpu.sync_copy(src_hbm.at[pl.ds(base, C)], src_v)
            pltpu.sync_copy(dst_hbm.at[pl.ds(base, C)], dst_v)

            def grp(j, _):
                rows = j * LN + iota
                s16 = src_v[pl.ds(j * LN, LN)]
                d16 = dst_v[pl.ds(j * LN, LN)]
                relx = plsc.load_gather(px_v, [d16]) - plsc.load_gather(px_v, [s16])
                rely = plsc.load_gather(py_v, [d16]) - plsc.load_gather(py_v, [s16])
                relz = plsc.load_gather(pz_v, [d16]) - plsc.load_gather(pz_v, [s16])
                r2 = relx * relx + rely * rely + relz * relz
                y = _rsqrt_nr(r2)
                d = r2 * y
                inv = jnp.minimum(y, 1e12)
                plsc.store_scatter(fbuf, [rows, fcols[0]], relx * inv)
                plsc.store_scatter(fbuf, [rows, fcols[1]], rely * inv)
                plsc.store_scatter(fbuf, [rows, fcols[2]], relz * inv)
                for kk in range(R):
                    dk = d - centers[kk]
                    plsc.store_scatter(fbuf, [rows, fcols[3 + kk]],
                                       jnp.exp(-width * dk * dk))
                plsc.store_scatter(fbuf, [rows, fcols[19]],
                                   zero16 + 1.0)
                return 0

            lax.fori_loop(0, C // LN, grp, 0)
            pltpu.sync_copy(fbuf, gacc.at[dst_v], add=True)

        def tbody(t, _):
            g = s + t * NS

            @pl.when(g < NCH)
            def _():
                chunk(g)

            return 0

        lax.fori_loop(0, MAXT, tbody, 0)
        plsc.subcore_barrier()

        def wrow(t, _):
            g = s + t * NS

            @pl.when(g < NRC)
            def _():
                pltpu.sync_copy(gacc.at[pl.ds(g * RC, RC)], fbuf.at[pl.ds(0, RC)])
                pltpu.sync_copy(fbuf.at[pl.ds(0, RC)],
                                out_hbm.at[c].at[pl.ds(g * RC, RC)])

            return 0

        lax.fori_loop(0, -(-NRC // NS), wrow, 0)

    return k(pos[:, 0].astype(jnp.float32), pos[:, 1].astype(jnp.float32),
             pos[:, 2].astype(jnp.float32), src, dst)


def _segsum(h, src, dst, N):
    E = src.shape[0]
    NC, NS = 2, 16
    EPC = E // NC
    C = 128
    NCH = EPC // C
    NPAIR = -(-NCH // NS) // 2 + 1
    RC = 80
    NRC = N // RC
    mesh = plsc.VectorSubcoreMesh(core_axis_name="c", subcore_axis_name="s", num_cores=2, num_subcores=16)

    @functools.partial(
        pl.kernel,
        out_type=jax.ShapeDtypeStruct((NC, N, H), jnp.float32),
        mesh=mesh,
        scratch_types=[
            pltpu.VMEM_SHARED((N, H), jnp.float32),
            pltpu.VMEM((C,), jnp.int32),
            pltpu.VMEM((C,), jnp.int32),
            pltpu.VMEM((C,), jnp.int32),
            pltpu.VMEM((C,), jnp.int32),
            pltpu.VMEM((C, H), jnp.float32),
            pltpu.VMEM((C, H), jnp.float32),
            pltpu.SemaphoreType.DMA,
            pltpu.SemaphoreType.DMA,
        ],
    )
    def k(h_hbm, src_hbm, dst_hbm, out_hbm, acc, srcA, dstA, srcB, dstB,
          rowsA, rowsB, semA, semB):
        c = lax.axis_index("c")
        s = lax.axis_index("s")

        zero16 = jnp.zeros((16,), jnp.float32)

        def zbody(r, _):
            for j in range(H // 16):
                rowsA[r, pl.ds(j * 16, 16)] = zero16
            return 0

        lax.fori_loop(0, C, zbody, 0)
        for t in range(-(-NRC // NS)):
            g = s + t * NS

            @pl.when(g < NRC)
            def _():
                pltpu.sync_copy(rowsA.at[pl.ds(0, RC)],
                                acc.at[pl.ds(g * RC, RC)])

        plsc.subcore_barrier()

        e0 = c * EPC

        def load(g, src_v, dst_v):
            base = e0 + g * C
            pltpu.sync_copy(src_hbm.at[pl.ds(base, C)], src_v)
            pltpu.sync_copy(dst_hbm.at[pl.ds(base, C)], dst_v)

        load(s, srcA, dstA)
        pltpu.async_copy(h_hbm.at[srcA], rowsA, semA)

        def pair(u, _):
            g0 = s + (2 * u) * NS
            g1 = s + (2 * u + 1) * NS
            g2 = s + (2 * u + 2) * NS

            @pl.when(g1 < NCH)
            def _():
                load(g1, srcB, dstB)
                pltpu.async_copy(h_hbm.at[srcB], rowsB, semB)

            @pl.when(g0 < NCH)
            def _():
                pltpu.make_async_copy(h_hbm.at[srcA], rowsA, semA).wait()
                pltpu.sync_copy(rowsA, acc.at[dstA], add=True)

            @pl.when(g2 < NCH)
            def _():
                load(g2, srcA, dstA)
                pltpu.async_copy(h_hbm.at[srcA], rowsA, semA)

            @pl.when(g1 < NCH)
            def _():
                pltpu.make_async_copy(h_hbm.at[srcB], rowsB, semB).wait()
                pltpu.sync_copy(rowsB, acc.at[dstB], add=True)

            return 0

        lax.fori_loop(0, NPAIR, pair, 0)
        plsc.subcore_barrier()

        for t in range(-(-NRC // NS)):
            g = s + t * NS

            @pl.when(g < NRC)
            def _():
                pltpu.sync_copy(acc.at[pl.ds(g * RC, RC)],
                                rowsA.at[pl.ds(0, RC)])
                pltpu.sync_copy(rowsA.at[pl.ds(0, RC)],
                                out_hbm.at[c].at[pl.ds(g * RC, RC)])

    return k(h, src, dst)


def kernel(x, pos, edge_index, W1, b1, W2, b2, ne_ln_g, ne_ln_b, convW, convB,
           conv_ln_g, conv_ln_b, ln_g, ln_b, Wg1, bg1, Wg2, bg2, Wc1, bc1,
           Wc2, bc2, Wp1, bp1, Wp2, bp2):
    N = x.shape[0]
    L = convW.shape[0]
    src = edge_index[0]
    dst = edge_index[1]

    Wgeo = jnp.concatenate(
        [convW[:, H:H + 19, :], convB[:, None, :],
         jnp.zeros((L, GEOF - 20, H), jnp.float32)], axis=1)
    Wh = convW[:, :H, :]

    h = _encoder(x, W1, b1, W2, b2, ne_ln_g, ne_ln_b)
    geo = _geo_aggregate(pos, src, dst, N)

    for i in range(L):
        seg2 = _segsum(h, src, dst, N)
        h = _layer_dense(seg2, geo, h, Wh[i], Wgeo[i], conv_ln_g[i],
                         conv_ln_b[i], ln_g[i], ln_b[i])

    return _readout(h, Wg1, bg1, Wg2, bg2, Wc1, bc1, Wc2, bc2,
                    Wp1, bp1, Wp2, bp2)


# speedup vs baseline: 6.4686x; 1.0386x over previous
import dataclasses
import functools

import jax
import jax.numpy as jnp
from jax import lax
from jax.experimental import pallas as pl
from jax.experimental.pallas import tpu as pltpu
from jax.experimental.pallas import tpu_sc as plsc

H = 128
GEOF = 128
GEOW = 32


def _sc_params():
    cp = pltpu.CompilerParams()
    if "needs_layout_passes" in pltpu.CompilerParams.__dataclass_fields__:
        cp = dataclasses.replace(cp, needs_layout_passes=False)
    return cp


def _silu(x):
    return x * jax.nn.sigmoid(x)


def _ln(x, g, b, eps=1e-5):
    m = jnp.mean(x, axis=-1, keepdims=True)
    v = jnp.mean((x - m) ** 2, axis=-1, keepdims=True)
    return (x - m) * jax.lax.rsqrt(v + eps) * g + b


def _enc_body(x_ref, w1_ref, b1_ref, w2_ref, b2_ref, g_ref, bb_ref, o_ref):
    h = jnp.dot(x_ref[...], w1_ref[...], preferred_element_type=jnp.float32)
    h = _silu(h + b1_ref[...])
    h = jnp.dot(h, w2_ref[...], preferred_element_type=jnp.float32) + b2_ref[...]
    o_ref[...] = _ln(h, g_ref[...], bb_ref[...])


def _encoder(x, W1, b1, W2, b2, g, b):
    N, D = x.shape
    BN = 1000
    grid = (N // BN,)
    return pl.pallas_call(
        _enc_body,
        grid=grid,
        in_specs=[
            pl.BlockSpec((BN, D), lambda i: (i, 0)),
            pl.BlockSpec((D, H), lambda i: (0, 0)),
            pl.BlockSpec((1, H), lambda i: (0, 0)),
            pl.BlockSpec((H, H), lambda i: (0, 0)),
            pl.BlockSpec((1, H), lambda i: (0, 0)),
            pl.BlockSpec((1, H), lambda i: (0, 0)),
            pl.BlockSpec((1, H), lambda i: (0, 0)),
        ],
        out_specs=pl.BlockSpec((BN, H), lambda i: (i, 0)),
        out_shape=jax.ShapeDtypeStruct((N, H), jnp.float32),
    )(x, W1, b1.reshape(1, H), W2, b2.reshape(1, H), g.reshape(1, H), b.reshape(1, H))


def _layer_body(seg_ref, geo_ref, h_ref, wh_ref, wg_ref, cg_ref, cb_ref,
                lg_ref, lb_ref, o_ref):
    seg = seg_ref[0] + seg_ref[1]
    geo = geo_ref[0] + geo_ref[1]
    agg = jnp.dot(seg, wh_ref[...], preferred_element_type=jnp.float32)
    agg = agg + jnp.dot(geo, wg_ref[...], preferred_element_type=jnp.float32)
    h_new = _ln(_silu(agg), cg_ref[...], cb_ref[...])
    o_ref[...] = _ln(h_ref[...] + h_new, lg_ref[...], lb_ref[...])


def _layer_dense(seg2, geo, h, Wh, Wgeo, cg, cb, lg, lb):
    N = h.shape[0]
    BN = 1000
    grid = (N // BN,)
    return pl.pallas_call(
        _layer_body,
        grid=grid,
        in_specs=[
            pl.BlockSpec((2, BN, H), lambda i: (0, i, 0)),
            pl.BlockSpec((2, BN, GEOW), lambda i: (0, i, 0)),
            pl.BlockSpec((BN, H), lambda i: (i, 0)),
            pl.BlockSpec((H, H), lambda i: (0, 0)),
            pl.BlockSpec((GEOW, H), lambda i: (0, 0)),
            pl.BlockSpec((1, H), lambda i: (0, 0)),
            pl.BlockSpec((1, H), lambda i: (0, 0)),
            pl.BlockSpec((1, H), lambda i: (0, 0)),
            pl.BlockSpec((1, H), lambda i: (0, 0)),
        ],
        out_specs=pl.BlockSpec((BN, H), lambda i: (i, 0)),
        out_shape=jax.ShapeDtypeStruct((N, H), jnp.float32),
    )(seg2, geo, h, Wh, Wgeo, cg.reshape(1, H), cb.reshape(1, H),
      lg.reshape(1, H), lb.reshape(1, H))


def _readout_body(h_ref, wg1_ref, bg1_ref, wg2_ref, bg2_ref,
                  wc1_ref, bc1_ref, wc2_ref, bc2_ref,
                  wp1_ref, bp1_ref, wp2_ref, bp2_ref,
                  logits_ref, proj_ref, emb_ref,
                  m_s, s_s, v_s):
    i = pl.program_id(0)
    nb = pl.num_programs(0)
    h = h_ref[...]
    z = jnp.dot(_silu(jnp.dot(h, wg1_ref[...],
                              preferred_element_type=jnp.float32) + bg1_ref[...]),
                wg2_ref[...], preferred_element_type=jnp.float32) + bg2_ref[0, 0]
    bm = jnp.max(z)

    @pl.when(i == 0)
    def _():
        m_s[...] = jnp.full_like(m_s, -jnp.inf)
        s_s[...] = jnp.zeros_like(s_s)
        v_s[...] = jnp.zeros_like(v_s)

    m_old = m_s[0, 0]
    m_new = jnp.maximum(m_old, bm)
    scale = jnp.exp(m_old - m_new)
    w = jnp.exp(z - m_new)
    s_s[...] = s_s[...] * scale + jnp.sum(w)
    v_s[...] = v_s[...] * scale + jnp.sum(h * w, axis=0, keepdims=True)
    m_s[...] = jnp.full_like(m_s, m_new)

    @pl.when(i == nb - 1)
    def _():
        emb = v_s[...] / s_s[0, 0]
        emb_ref[...] = emb
        c = jnp.dot(_silu(jnp.dot(emb, wc1_ref[...],
                                  preferred_element_type=jnp.float32) + bc1_ref[...]),
                    wc2_ref[...], preferred_element_type=jnp.float32) + bc2_ref[...]
        logits_ref[...] = c
        p = jnp.dot(_silu(jnp.dot(emb, wp1_ref[...],
                                  preferred_element_type=jnp.float32) + bp1_ref[...]),
                    wp2_ref[...], preferred_element_type=jnp.float32) + bp2_ref[...]
        nrm = jnp.maximum(jnp.sqrt(jnp.sum(p * p)), 1e-12)
        proj_ref[...] = p / nrm


def _readout(h, Wg1, bg1, Wg2, bg2, Wc1, bc1, Wc2, bc2, Wp1, bp1, Wp2, bp2):
    N = h.shape[0]
    BN = 1000
    grid = (N // BN,)
    Hq = Wg1.shape[1]
    Hc = Wc1.shape[1]
    C = Wc2.shape[1]
    P = Wp2.shape[1]
    full = lambda r, c: pl.BlockSpec((r, c), lambda i: (0, 0))
    return pl.pallas_call(
        _readout_body,
        grid=grid,
        in_specs=[
            pl.BlockSpec((BN, H), lambda i: (i, 0)),
            full(H, Hq), full(1, Hq), full(Hq, 1), full(1, 1),
            full(H, Hc), full(1, Hc), full(Hc, C), full(1, C),
            full(H, H), full(1, H), full(H, P), full(1, P),
        ],
        out_specs=[full(1, C), full(1, P), full(1, H)],
        out_shape=[
            jax.ShapeDtypeStruct((1, C), jnp.float32),
            jax.ShapeDtypeStruct((1, P), jnp.float32),
            jax.ShapeDtypeStruct((1, H), jnp.float32),
        ],
        scratch_shapes=[
            pltpu.VMEM((1, 128), jnp.float32),
            pltpu.VMEM((1, 128), jnp.float32),
            pltpu.VMEM((1, H), jnp.float32),
        ],
    )(h, Wg1, bg1.reshape(1, Hq), Wg2, bg2.reshape(1, 1),
      Wc1, bc1.reshape(1, Hc), Wc2, bc2.reshape(1, C),
      Wp1, bp1.reshape(1, H), Wp2, bp2.reshape(1, P))


def _rsqrt_nr(r2):
    bits = plsc.bitcast(r2, jnp.int32)
    y = plsc.bitcast(jnp.int32(0x5F3759DF) - (bits >> 1), jnp.float32)
    for _ in range(3):
        y = y * (1.5 - 0.5 * r2 * y * y)
    return y


def _geo_aggregate(pos, src, dst, N):
    E = src.shape[0]
    NC, NS, LN = 2, 16, 16
    C = 64
    NCH = E // NC // C
    MAXT = -(-NCH // NS)
    RC = 40
    NRC = N // RC
    cutoff, R = 10.0, 16
    width = 0.5 * (cutoff / R) ** 2
    centers = [cutoff * kk / (R - 1) for kk in range(R)]
    mesh = plsc.VectorSubcoreMesh(core_axis_name="c", subcore_axis_name="s", num_cores=2, num_subcores=16)

    @functools.partial(
        pl.kernel,
        out_type=jax.ShapeDtypeStruct((NC, N, GEOF), jnp.float32),
        mesh=mesh,
        compiler_params=_sc_params(),
        scratch_types=[
            pltpu.VMEM_SHARED((N, GEOF), jnp.float32),
            pltpu.VMEM((N,), jnp.float32),
            pltpu.VMEM((N,), jnp.float32),
            pltpu.VMEM((N,), jnp.float32),
            pltpu.VMEM((C,), jnp.int32),
            pltpu.VMEM((C,), jnp.int32),
            pltpu.VMEM((C, GEOF), jnp.float32),
        ],
    )
    def k(px_hbm, py_hbm, pz_hbm, src_hbm, dst_hbm, out_hbm, gacc,
          px_v, py_v, pz_v, src_v, dst_v, fbuf):
        c = lax.axis_index("c")
        s = lax.axis_index("s")
        zero16 = jnp.zeros((LN,), jnp.float32)

        pltpu.sync_copy(px_hbm, px_v)
        pltpu.sync_copy(py_hbm, py_v)
        pltpu.sync_copy(pz_hbm, pz_v)

        def zbody(r, _):
            for j in range(GEOF // LN):
                fbuf[r, pl.ds(j * LN, LN)] = zero16
            return 0

        lax.fori_loop(0, C, zbody, 0)

        def zrow(t, _):
            g = s + t * NS

            @pl.when(g < NRC)
            def _():
                pltpu.sync_copy(fbuf.at[pl.ds(0, RC)], gacc.at[pl.ds(g * RC, RC)])

            return 0

        lax.fori_loop(0, -(-NRC // NS), zrow, 0)
        plsc.subcore_barrier()

        iota = lax.iota(jnp.int32, LN)
        fcols = [jnp.full((LN,), f, jnp.int32) for f in range(20)]

        def chunk(g):
            base = c * (E // NC) + g * C
            pltpu.sync_copy(src_hbm.at[pl.ds(base, C)], src_v)
            pltpu.sync_copy(dst_hbm.at[pl.ds(base, C)], dst_v)

            def grp(j, _):
                rows = j * LN + iota
                s16 = src_v[pl.ds(j * LN, LN)]
                d16 = dst_v[pl.ds(j * LN, LN)]
                relx = plsc.load_gather(px_v, [d16]) - plsc.load_gather(px_v, [s16])
                rely = plsc.load_gather(py_v, [d16]) - plsc.load_gather(py_v, [s16])
                relz = plsc.load_gather(pz_v, [d16]) - plsc.load_gather(pz_v, [s16])
                r2 = relx * relx + rely * rely + relz * relz
                y = _rsqrt_nr(r2)
                d = r2 * y
                inv = jnp.minimum(y, 1e12)
                plsc.store_scatter(fbuf, [rows, fcols[0]], relx * inv)
                plsc.store_scatter(fbuf, [rows, fcols[1]], rely * inv)
                plsc.store_scatter(fbuf, [rows, fcols[2]], relz * inv)
                for kk in range(R):
                    dk = d - centers[kk]
                    plsc.store_scatter(fbuf, [rows, fcols[3 + kk]],
                                       jnp.exp(-width * dk * dk))
                plsc.store_scatter(fbuf, [rows, fcols[19]],
                                   zero16 + 1.0)
                return 0

            lax.fori_loop(0, C // LN, grp, 0)
            pltpu.sync_copy(fbuf, gacc.at[dst_v], add=True)

        def tbody(t, _):
            g = s + t * NS

            @pl.when(g < NCH)
            def _():
                chunk(g)

            return 0

        lax.fori_loop(0, MAXT, tbody, 0)
        plsc.subcore_barrier()

        def wrow(t, _):
            g = s + t * NS

            @pl.when(g < NRC)
            def _():
                pltpu.sync_copy(gacc.at[pl.ds(g * RC, RC)], fbuf.at[pl.ds(0, RC)])
                pltpu.sync_copy(fbuf.at[pl.ds(0, RC)],
                                out_hbm.at[c].at[pl.ds(g * RC, RC)])

            return 0

        lax.fori_loop(0, -(-NRC // NS), wrow, 0)

    return k(pos[:, 0].astype(jnp.float32), pos[:, 1].astype(jnp.float32),
             pos[:, 2].astype(jnp.float32), src, dst)


def _segsum(h, src, dst, N):
    E = src.shape[0]
    NC, NS = 2, 16
    EPC = E // NC
    C = 160
    NCH = EPC // C
    NPAIR = -(-NCH // NS) // 2 + 1
    RC = 80
    NRC = N // RC
    mesh = plsc.VectorSubcoreMesh(core_axis_name="c", subcore_axis_name="s", num_cores=2, num_subcores=16)

    @functools.partial(
        pl.kernel,
        out_type=jax.ShapeDtypeStruct((NC, N, H), jnp.float32),
        mesh=mesh,
        scratch_types=[
            pltpu.VMEM_SHARED((N, H), jnp.float32),
            pltpu.VMEM((C,), jnp.int32),
            pltpu.VMEM((C,), jnp.int32),
            pltpu.VMEM((C,), jnp.int32),
            pltpu.VMEM((C,), jnp.int32),
            pltpu.VMEM((C, H), jnp.float32),
            pltpu.VMEM((C, H), jnp.float32),
            pltpu.SemaphoreType.DMA,
            pltpu.SemaphoreType.DMA,
        ],
    )
    def k(h_hbm, src_hbm, dst_hbm, out_hbm, acc, srcA, dstA, srcB, dstB,
          rowsA, rowsB, semA, semB):
        c = lax.axis_index("c")
        s = lax.axis_index("s")

        zero16 = jnp.zeros((16,), jnp.float32)

        def zbody(r, _):
            for j in range(H // 16):
                rowsA[r, pl.ds(j * 16, 16)] = zero16
            return 0

        lax.fori_loop(0, C, zbody, 0)
        for t in range(-(-NRC // NS)):
            g = s + t * NS

            @pl.when(g < NRC)
            def _():
                pltpu.sync_copy(rowsA.at[pl.ds(0, RC)],
                                acc.at[pl.ds(g * RC, RC)])

        plsc.subcore_barrier()

        e0 = c * EPC

        def load(g, src_v, dst_v):
            base = e0 + g * C
            pltpu.sync_copy(src_hbm.at[pl.ds(base, C)], src_v)
            pltpu.sync_copy(dst_hbm.at[pl.ds(base, C)], dst_v)

        load(s, srcA, dstA)
        pltpu.async_copy(h_hbm.at[srcA], rowsA, semA)

        def pair(u, _):
            g0 = s + (2 * u) * NS
            g1 = s + (2 * u + 1) * NS
            g2 = s + (2 * u + 2) * NS

            @pl.when(g1 < NCH)
            def _():
                load(g1, srcB, dstB)
                pltpu.async_copy(h_hbm.at[srcB], rowsB, semB)

            @pl.when(g0 < NCH)
            def _():
                pltpu.make_async_copy(h_hbm.at[srcA], rowsA, semA).wait()
                pltpu.sync_copy(rowsA, acc.at[dstA], add=True)

            @pl.when(g2 < NCH)
            def _():
                load(g2, srcA, dstA)
                pltpu.async_copy(h_hbm.at[srcA], rowsA, semA)

            @pl.when(g1 < NCH)
            def _():
                pltpu.make_async_copy(h_hbm.at[srcB], rowsB, semB).wait()
                pltpu.sync_copy(rowsB, acc.at[dstB], add=True)

            return 0

        lax.fori_loop(0, NPAIR, pair, 0)
        plsc.subcore_barrier()

        for t in range(-(-NRC // NS)):
            g = s + t * NS

            @pl.when(g < NRC)
            def _():
                pltpu.sync_copy(acc.at[pl.ds(g * RC, RC)],
                                rowsA.at[pl.ds(0, RC)])
                pltpu.sync_copy(rowsA.at[pl.ds(0, RC)],
                                out_hbm.at[c].at[pl.ds(g * RC, RC)])

    return k(h, src, dst)


def kernel(x, pos, edge_index, W1, b1, W2, b2, ne_ln_g, ne_ln_b, convW, convB,
           conv_ln_g, conv_ln_b, ln_g, ln_b, Wg1, bg1, Wg2, bg2, Wc1, bc1,
           Wc2, bc2, Wp1, bp1, Wp2, bp2):
    N = x.shape[0]
    L = convW.shape[0]
    src = edge_index[0]
    dst = edge_index[1]

    Wgeo = jnp.concatenate(
        [convW[:, H:H + 19, :], convB[:, None, :],
         jnp.zeros((L, GEOW - 20, H), jnp.float32)], axis=1)
    Wh = convW[:, :H, :]

    h = _encoder(x, W1, b1, W2, b2, ne_ln_g, ne_ln_b)
    geo = _geo_aggregate(pos, src, dst, N)[:, :, :GEOW]

    for i in range(L):
        seg2 = _segsum(h, src, dst, N)
        h = _layer_dense(seg2, geo, h, Wh[i], Wgeo[i], conv_ln_g[i],
                         conv_ln_b[i], ln_g[i], ln_b[i])

    return _readout(h, Wg1, bg1, Wg2, bg2, Wc1, bc1, Wc2, bc2,
                    Wp1, bp1, Wp2, bp2)


# speedup vs baseline: 6.9907x; 1.0807x over previous
import dataclasses
import functools

import jax
import jax.numpy as jnp
from jax import lax
from jax.experimental import pallas as pl
from jax.experimental.pallas import tpu as pltpu
from jax.experimental.pallas import tpu_sc as plsc

H = 128
GEOF = 128
GEOW = 32


def _sc_params():
    cp = pltpu.CompilerParams()
    if "needs_layout_passes" in pltpu.CompilerParams.__dataclass_fields__:
        cp = dataclasses.replace(cp, needs_layout_passes=False)
    return cp


def _silu(x):
    return x * jax.nn.sigmoid(x)


def _ln(x, g, b, eps=1e-5):
    m = jnp.mean(x, axis=-1, keepdims=True)
    v = jnp.mean((x - m) ** 2, axis=-1, keepdims=True)
    return (x - m) * jax.lax.rsqrt(v + eps) * g + b


def _enc_body(x_ref, w1_ref, b1_ref, w2_ref, b2_ref, g_ref, bb_ref, o_ref):
    h = jnp.dot(x_ref[...], w1_ref[...], preferred_element_type=jnp.float32)
    h = _silu(h + b1_ref[...])
    h = jnp.dot(h, w2_ref[...], preferred_element_type=jnp.float32) + b2_ref[...]
    o_ref[...] = _ln(h, g_ref[...], bb_ref[...])


def _encoder(x, W1, b1, W2, b2, g, b):
    N, D = x.shape
    BN = 1000
    grid = (N // BN,)
    return pl.pallas_call(
        _enc_body,
        grid=grid,
        in_specs=[
            pl.BlockSpec((BN, D), lambda i: (i, 0)),
            pl.BlockSpec((D, H), lambda i: (0, 0)),
            pl.BlockSpec((1, H), lambda i: (0, 0)),
            pl.BlockSpec((H, H), lambda i: (0, 0)),
            pl.BlockSpec((1, H), lambda i: (0, 0)),
            pl.BlockSpec((1, H), lambda i: (0, 0)),
            pl.BlockSpec((1, H), lambda i: (0, 0)),
        ],
        out_specs=pl.BlockSpec((BN, H), lambda i: (i, 0)),
        out_shape=jax.ShapeDtypeStruct((N, H), jnp.float32),
    )(x, W1, b1.reshape(1, H), W2, b2.reshape(1, H), g.reshape(1, H), b.reshape(1, H))


def _layer_body(seg_ref, geo_ref, h_ref, wh_ref, wg_ref, cg_ref, cb_ref,
                lg_ref, lb_ref, o_ref):
    seg = seg_ref[0] + seg_ref[1]
    geo = geo_ref[0] + geo_ref[1]
    agg = jnp.dot(seg, wh_ref[...], preferred_element_type=jnp.float32)
    agg = agg + jnp.dot(geo, wg_ref[...], preferred_element_type=jnp.float32)
    h_new = _ln(_silu(agg), cg_ref[...], cb_ref[...])
    o_ref[...] = _ln(h_ref[...] + h_new, lg_ref[...], lb_ref[...])


def _layer_dense(seg2, geo, h, Wh, Wgeo, cg, cb, lg, lb):
    N = h.shape[0]
    BN = 1000
    grid = (N // BN,)
    return pl.pallas_call(
        _layer_body,
        grid=grid,
        in_specs=[
            pl.BlockSpec((2, BN, H), lambda i: (0, i, 0)),
            pl.BlockSpec((2, BN, GEOW), lambda i: (0, i, 0)),
            pl.BlockSpec((BN, H), lambda i: (i, 0)),
            pl.BlockSpec((H, H), lambda i: (0, 0)),
            pl.BlockSpec((GEOW, H), lambda i: (0, 0)),
            pl.BlockSpec((1, H), lambda i: (0, 0)),
            pl.BlockSpec((1, H), lambda i: (0, 0)),
            pl.BlockSpec((1, H), lambda i: (0, 0)),
            pl.BlockSpec((1, H), lambda i: (0, 0)),
        ],
        out_specs=pl.BlockSpec((BN, H), lambda i: (i, 0)),
        out_shape=jax.ShapeDtypeStruct((N, H), jnp.float32),
    )(seg2, geo, h, Wh, Wgeo, cg.reshape(1, H), cb.reshape(1, H),
      lg.reshape(1, H), lb.reshape(1, H))


def _readout_body(h_ref, wg1_ref, bg1_ref, wg2_ref, bg2_ref,
                  wc1_ref, bc1_ref, wc2_ref, bc2_ref,
                  wp1_ref, bp1_ref, wp2_ref, bp2_ref,
                  logits_ref, proj_ref, emb_ref,
                  m_s, s_s, v_s):
    i = pl.program_id(0)
    nb = pl.num_programs(0)
    h = h_ref[...]
    z = jnp.dot(_silu(jnp.dot(h, wg1_ref[...],
                              preferred_element_type=jnp.float32) + bg1_ref[...]),
                wg2_ref[...], preferred_element_type=jnp.float32) + bg2_ref[0, 0]
    bm = jnp.max(z)

    @pl.when(i == 0)
    def _():
        m_s[...] = jnp.full_like(m_s, -jnp.inf)
        s_s[...] = jnp.zeros_like(s_s)
        v_s[...] = jnp.zeros_like(v_s)

    m_old = m_s[0, 0]
    m_new = jnp.maximum(m_old, bm)
    scale = jnp.exp(m_old - m_new)
    w = jnp.exp(z - m_new)
    s_s[...] = s_s[...] * scale + jnp.sum(w)
    v_s[...] = v_s[...] * scale + jnp.sum(h * w, axis=0, keepdims=True)
    m_s[...] = jnp.full_like(m_s, m_new)

    @pl.when(i == nb - 1)
    def _():
        emb = v_s[...] / s_s[0, 0]
        emb_ref[...] = emb
        c = jnp.dot(_silu(jnp.dot(emb, wc1_ref[...],
                                  preferred_element_type=jnp.float32) + bc1_ref[...]),
                    wc2_ref[...], preferred_element_type=jnp.float32) + bc2_ref[...]
        logits_ref[...] = c
        p = jnp.dot(_silu(jnp.dot(emb, wp1_ref[...],
                                  preferred_element_type=jnp.float32) + bp1_ref[...]),
                    wp2_ref[...], preferred_element_type=jnp.float32) + bp2_ref[...]
        nrm = jnp.maximum(jnp.sqrt(jnp.sum(p * p)), 1e-12)
        proj_ref[...] = p / nrm


def _readout(h, Wg1, bg1, Wg2, bg2, Wc1, bc1, Wc2, bc2, Wp1, bp1, Wp2, bp2):
    N = h.shape[0]
    BN = 1000
    grid = (N // BN,)
    Hq = Wg1.shape[1]
    Hc = Wc1.shape[1]
    C = Wc2.shape[1]
    P = Wp2.shape[1]
    full = lambda r, c: pl.BlockSpec((r, c), lambda i: (0, 0))
    return pl.pallas_call(
        _readout_body,
        grid=grid,
        in_specs=[
            pl.BlockSpec((BN, H), lambda i: (i, 0)),
            full(H, Hq), full(1, Hq), full(Hq, 1), full(1, 1),
            full(H, Hc), full(1, Hc), full(Hc, C), full(1, C),
            full(H, H), full(1, H), full(H, P), full(1, P),
        ],
        out_specs=[full(1, C), full(1, P), full(1, H)],
        out_shape=[
            jax.ShapeDtypeStruct((1, C), jnp.float32),
            jax.ShapeDtypeStruct((1, P), jnp.float32),
            jax.ShapeDtypeStruct((1, H), jnp.float32),
        ],
        scratch_shapes=[
            pltpu.VMEM((1, 128), jnp.float32),
            pltpu.VMEM((1, 128), jnp.float32),
            pltpu.VMEM((1, H), jnp.float32),
        ],
    )(h, Wg1, bg1.reshape(1, Hq), Wg2, bg2.reshape(1, 1),
      Wc1, bc1.reshape(1, Hc), Wc2, bc2.reshape(1, C),
      Wp1, bp1.reshape(1, H), Wp2, bp2.reshape(1, P))


def _rsqrt_nr(r2):
    bits = plsc.bitcast(r2, jnp.int32)
    y = plsc.bitcast(jnp.int32(0x5F3759DF) - (bits >> 1), jnp.float32)
    for _ in range(3):
        y = y * (1.5 - 0.5 * r2 * y * y)
    return y


def _geo_aggregate(pos, src, dst, N):
    E = src.shape[0]
    NC, NS, LN = 2, 16, 16
    C = 128
    NCH = E // NC // C
    MAXT = -(-NCH // NS)
    RC = 80
    NRC = N // RC
    cutoff, R = 10.0, 16
    width = 0.5 * (cutoff / R) ** 2
    centers = [cutoff * kk / (R - 1) for kk in range(R)]
    mesh = plsc.VectorSubcoreMesh(core_axis_name="c", subcore_axis_name="s", num_cores=2, num_subcores=16)

    @functools.partial(
        pl.kernel,
        out_type=jax.ShapeDtypeStruct((NC, N, GEOF), jnp.float32),
        mesh=mesh,
        compiler_params=_sc_params(),
        scratch_types=[
            pltpu.VMEM_SHARED((N, GEOF), jnp.float32),
            pltpu.VMEM((N,), jnp.float32),
            pltpu.VMEM((N,), jnp.float32),
            pltpu.VMEM((N,), jnp.float32),
            pltpu.VMEM((2, C), jnp.int32),
            pltpu.VMEM((C, GEOF), jnp.float32),
        ],
    )
    def k(px_hbm, py_hbm, pz_hbm, idx_hbm, out_hbm, gacc,
          px_v, py_v, pz_v, idx_v, fbuf):
        c = lax.axis_index("c")
        s = lax.axis_index("s")
        zero16 = jnp.zeros((LN,), jnp.float32)

        pltpu.sync_copy(px_hbm, px_v)
        pltpu.sync_copy(py_hbm, py_v)
        pltpu.sync_copy(pz_hbm, pz_v)

        def zbody(r, _):
            for j in range(GEOF // LN):
                fbuf[r, pl.ds(j * LN, LN)] = zero16
            return 0

        lax.fori_loop(0, C, zbody, 0)

        def zrow(t, _):
            g = s + t * NS

            @pl.when(g < NRC)
            def _():
                pltpu.sync_copy(fbuf.at[pl.ds(0, RC)], gacc.at[pl.ds(g * RC, RC)])

            return 0

        lax.fori_loop(0, -(-NRC // NS), zrow, 0)
        plsc.subcore_barrier()

        iota = lax.iota(jnp.int32, LN)
        fcols = [jnp.full((LN,), f, jnp.int32) for f in range(20)]

        def chunk(g):
            pltpu.sync_copy(idx_hbm.at[c * NCH + g], idx_v)

            def grp(j, _):
                rows = j * LN + iota
                s16 = idx_v[0, pl.ds(j * LN, LN)]
                d16 = idx_v[1, pl.ds(j * LN, LN)]
                relx = plsc.load_gather(px_v, [d16]) - plsc.load_gather(px_v, [s16])
                rely = plsc.load_gather(py_v, [d16]) - plsc.load_gather(py_v, [s16])
                relz = plsc.load_gather(pz_v, [d16]) - plsc.load_gather(pz_v, [s16])
                r2 = relx * relx + rely * rely + relz * relz
                y = _rsqrt_nr(r2)
                d = r2 * y
                inv = jnp.minimum(y, 1e12)
                plsc.store_scatter(fbuf, [rows, fcols[0]], relx * inv)
                plsc.store_scatter(fbuf, [rows, fcols[1]], rely * inv)
                plsc.store_scatter(fbuf, [rows, fcols[2]], relz * inv)
                for kk in range(R):
                    dk = d - centers[kk]
                    plsc.store_scatter(fbuf, [rows, fcols[3 + kk]],
                                       jnp.exp(-width * dk * dk))
                plsc.store_scatter(fbuf, [rows, fcols[19]],
                                   zero16 + 1.0)
                return 0

            lax.fori_loop(0, C // LN, grp, 0)
            pltpu.sync_copy(fbuf, gacc.at[idx_v.at[1]], add=True)

        def tbody(t, _):
            g = s + t * NS

            @pl.when(g < NCH)
            def _():
                chunk(g)

            return 0

        lax.fori_loop(0, MAXT, tbody, 0)
        plsc.subcore_barrier()

        def wrow(t, _):
            g = s + t * NS

            @pl.when(g < NRC)
            def _():
                pltpu.sync_copy(gacc.at[pl.ds(g * RC, RC)], fbuf.at[pl.ds(0, RC)])
                pltpu.sync_copy(fbuf.at[pl.ds(0, RC)],
                                out_hbm.at[c].at[pl.ds(g * RC, RC)])

            return 0

        lax.fori_loop(0, -(-NRC // NS), wrow, 0)

    idxpack = jnp.stack([src.reshape(-1, C), dst.reshape(-1, C)], axis=1)
    return k(pos[:, 0].astype(jnp.float32), pos[:, 1].astype(jnp.float32),
             pos[:, 2].astype(jnp.float32), idxpack)


def _segsum(h, src, dst, N):
    E = src.shape[0]
    NC, NS = 2, 16
    EPC = E // NC
    C = 160
    NCH = EPC // C
    NPAIR = -(-NCH // NS) // 2 + 1
    RC = 80
    NRC = N // RC
    mesh = plsc.VectorSubcoreMesh(core_axis_name="c", subcore_axis_name="s", num_cores=2, num_subcores=16)

    @functools.partial(
        pl.kernel,
        out_type=jax.ShapeDtypeStruct((NC, N, H), jnp.float32),
        mesh=mesh,
        scratch_types=[
            pltpu.VMEM_SHARED((N, H), jnp.float32),
            pltpu.VMEM((C,), jnp.int32),
            pltpu.VMEM((C,), jnp.int32),
            pltpu.VMEM((C,), jnp.int32),
            pltpu.VMEM((C,), jnp.int32),
            pltpu.VMEM((C, H), jnp.float32),
            pltpu.VMEM((C, H), jnp.float32),
            pltpu.SemaphoreType.DMA,
            pltpu.SemaphoreType.DMA,
        ],
    )
    def k(h_hbm, src_hbm, dst_hbm, out_hbm, acc, srcA, dstA, srcB, dstB,
          rowsA, rowsB, semA, semB):
        c = lax.axis_index("c")
        s = lax.axis_index("s")

        zero16 = jnp.zeros((16,), jnp.float32)

        def zbody(r, _):
            for j in range(H // 16):
                rowsA[r, pl.ds(j * 16, 16)] = zero16
            return 0

        lax.fori_loop(0, C, zbody, 0)
        for t in range(-(-NRC // NS)):
            g = s + t * NS

            @pl.when(g < NRC)
            def _():
                pltpu.sync_copy(rowsA.at[pl.ds(0, RC)],
                                acc.at[pl.ds(g * RC, RC)])

        plsc.subcore_barrier()

        e0 = c * EPC

        def load(g, src_v, dst_v):
            base = e0 + g * C
            pltpu.sync_copy(src_hbm.at[pl.ds(base, C)], src_v)
            pltpu.sync_copy(dst_hbm.at[pl.ds(base, C)], dst_v)

        load(s, srcA, dstA)
        pltpu.async_copy(h_hbm.at[srcA], rowsA, semA)

        def pair(u, _):
            g0 = s + (2 * u) * NS
            g1 = s + (2 * u + 1) * NS
            g2 = s + (2 * u + 2) * NS

            @pl.when(g1 < NCH)
            def _():
                load(g1, srcB, dstB)
                pltpu.async_copy(h_hbm.at[srcB], rowsB, semB)

            @pl.when(g0 < NCH)
            def _():
                pltpu.make_async_copy(h_hbm.at[srcA], rowsA, semA).wait()
                pltpu.sync_copy(rowsA, acc.at[dstA], add=True)

            @pl.when(g2 < NCH)
            def _():
                load(g2, srcA, dstA)
                pltpu.async_copy(h_hbm.at[srcA], rowsA, semA)

            @pl.when(g1 < NCH)
            def _():
                pltpu.make_async_copy(h_hbm.at[srcB], rowsB, semB).wait()
                pltpu.sync_copy(rowsB, acc.at[dstB], add=True)

            return 0

        lax.fori_loop(0, NPAIR, pair, 0)
        plsc.subcore_barrier()

        for t in range(-(-NRC // NS)):
            g = s + t * NS

            @pl.when(g < NRC)
            def _():
                pltpu.sync_copy(acc.at[pl.ds(g * RC, RC)],
                                rowsA.at[pl.ds(0, RC)])
                pltpu.sync_copy(rowsA.at[pl.ds(0, RC)],
                                out_hbm.at[c].at[pl.ds(g * RC, RC)])

    return k(h, src, dst)


def kernel(x, pos, edge_index, W1, b1, W2, b2, ne_ln_g, ne_ln_b, convW, convB,
           conv_ln_g, conv_ln_b, ln_g, ln_b, Wg1, bg1, Wg2, bg2, Wc1, bc1,
           Wc2, bc2, Wp1, bp1, Wp2, bp2):
    N = x.shape[0]
    L = convW.shape[0]
    src = edge_index[0]
    dst = edge_index[1]

    Wgeo = jnp.concatenate(
        [convW[:, H:H + 19, :], convB[:, None, :],
         jnp.zeros((L, GEOW - 20, H), jnp.float32)], axis=1)
    Wh = convW[:, :H, :]

    h = _encoder(x, W1, b1, W2, b2, ne_ln_g, ne_ln_b)
    geo = _geo_aggregate(pos, src, dst, N)[:, :, :GEOW]

    for i in range(L):
        seg2 = _segsum(h, src, dst, N)
        h = _layer_dense(seg2, geo, h, Wh[i], Wgeo[i], conv_ln_g[i],
                         conv_ln_b[i], ln_g[i], ln_b[i])

    return _readout(h, Wg1, bg1, Wg2, bg2, Wc1, bc1, Wc2, bc2,
                    Wp1, bp1, Wp2, bp2)
